# Initial kernel scaffold; baseline (speedup 1.0000x reference)
import jax
import jax.numpy as jnp
from jax.experimental import pallas as pl


def kernel(node_features, edge_index, edge_attr, W_embed, b_embed, Wm1_0, bm1_0, Wm2_0, bm2_0, Wu_0, bu_0, g_0, be_0, Wm1_1, bm1_1, Wm2_1, bm2_1, Wu_1, bu_1, g_1, be_1, Wm1_2, bm1_2, Wm2_2, bm2_2, Wu_2, bu_2, g_2, be_2, Wp1, bp1, Wp2, bp2, Wp3, bp3):
    raise NotImplementedError("write your pallas kernel here")


# speedup vs baseline: 4.2311x; 4.2311x over previous
import functools

import jax
import jax.numpy as jnp
from jax import lax
from jax.experimental import pallas as pl
from jax.experimental.pallas import tpu as pltpu
from jax.experimental.pallas import tpu_sc as plsc

N = 50000
E = 800000
D = 32
H = 32

NC = 2
NS = 16
LANES = 16

K = 128
NCHUNK = E // K
NPAD = 51200
RPS = NPAD // NS
ZCH = 200

_mesh = plsc.VectorSubcoreMesh(core_axis_name="c", subcore_axis_name="s")
_sc_params = pltpu.CompilerParams(use_tc_tiling_on_sc=False)


def _cnt_body(dst_hbm, out_hbm, dstv, onesv, zb, cnt_sh, sem):
    c = lax.axis_index("c")
    s = lax.axis_index("s")
    wid = s * NC + c

    @pl.loop(0, 8)
    def _fill_ones(i):
        onesv[pl.ds(i * 16, 16)] = jnp.full((16,), 1.0, jnp.float32)

    @pl.loop(0, 50)
    def _fill_zeros(i):
        zb[pl.ds(i * 16, 16)] = jnp.zeros((16,), jnp.float32)

    base = s * RPS

    @pl.loop(0, RPS // 800)
    def _zero(k):
        pltpu.sync_copy(zb, cnt_sh.at[pl.ds(base + k * 800, 800)])

    plsc.subcore_barrier()

    @pl.loop(wid, NCHUNK, step=NC * NS)
    def _scatter(t):
        eb = t * K
        pltpu.sync_copy(dst_hbm.at[pl.ds(eb, K)], dstv)
        pltpu.sync_copy(onesv, cnt_sh.at[dstv], add=True)

    plsc.subcore_barrier()

    @pl.loop(0, RPS // 800)
    def _drain(k):
        o = base + k * 800
        pltpu.sync_copy(cnt_sh.at[pl.ds(o, 800)], zb)
        pltpu.sync_copy(zb, out_hbm.at[pl.ds(c * NPAD + o, 800)])


_count_kernel = functools.partial(
    pl.kernel,
    out_type=jax.ShapeDtypeStruct((NC * NPAD,), jnp.float32),
    mesh=_mesh,
    compiler_params=_sc_params,
    scratch_types=[
        pltpu.VMEM((K,), jnp.int32),
        pltpu.VMEM((K,), jnp.float32),
        pltpu.VMEM((800,), jnp.float32),
        pltpu.VMEM_SHARED((NPAD,), jnp.float32),
        pltpu.SemaphoreType.DMA,
    ],
)(_cnt_body)


def _edge_compute(asv, bdv, cv, hv):
    @pl.loop(0, K)
    def _rows(i):
        for j in (0, 16):
            hv[i, pl.ds(j, 16)] = jnp.maximum(
                asv[i, pl.ds(j, 16)] + bdv[i, pl.ds(j, 16)] + cv[i, pl.ds(j, 16)],
                0.0,
            )


def _agg_body(a_hbm, b_hbm, c_hbm, src_hbm, dst_hbm, out_hbm,
              srcv, dstv, asv, bdv, cv, hv, zb, agg_sh, sem0, sem1, sem2):
    c = lax.axis_index("c")
    s = lax.axis_index("s")
    wid = s * NC + c

    @pl.loop(0, ZCH)
    def _fill_zeros(r):
        zb[r, pl.ds(0, 16)] = jnp.zeros((16,), jnp.float32)
        zb[r, pl.ds(16, 16)] = jnp.zeros((16,), jnp.float32)

    rbase = s * RPS

    @pl.loop(0, RPS // ZCH)
    def _zero(k):
        pltpu.sync_copy(zb, agg_sh.at[pl.ds(rbase + k * ZCH, ZCH)])

    plsc.subcore_barrier()

    @pl.loop(wid, NCHUNK, step=NC * NS)
    def _chunk(t):
        eb = t * K
        pltpu.sync_copy(src_hbm.at[pl.ds(eb, K)], srcv)
        pltpu.sync_copy(dst_hbm.at[pl.ds(eb, K)], dstv)
        ca = pltpu.async_copy(a_hbm.at[srcv], asv, sem0)
        cb = pltpu.async_copy(b_hbm.at[dstv], bdv, sem1)
        cc = pltpu.async_copy(c_hbm.at[pl.ds(eb, K)], cv, sem2)
        ca.wait()
        cb.wait()
        cc.wait()
        _edge_compute(asv, bdv, cv, hv)
        pltpu.sync_copy(hv, agg_sh.at[dstv], add=True)

    plsc.subcore_barrier()

    @pl.loop(0, RPS // ZCH)
    def _drain(k):
        o = rbase + k * ZCH
        pltpu.sync_copy(agg_sh.at[pl.ds(o, ZCH)], zb)
        pltpu.sync_copy(zb, out_hbm.at[c, pl.ds(o, ZCH)])


_agg_kernel = functools.partial(
    pl.kernel,
    out_type=jax.ShapeDtypeStruct((NC, NPAD, H), jnp.float32),
    mesh=_mesh,
    compiler_params=_sc_params,
    scratch_types=[
        pltpu.VMEM((K,), jnp.int32),
        pltpu.VMEM((K,), jnp.int32),
        pltpu.VMEM((K, D), jnp.float32),
        pltpu.VMEM((K, D), jnp.float32),
        pltpu.VMEM((K, H), jnp.float32),
        pltpu.VMEM((K, H), jnp.float32),
        pltpu.VMEM((ZCH, H), jnp.float32),
        pltpu.VMEM_SHARED((NPAD, H), jnp.float32),
        pltpu.SemaphoreType.DMA,
        pltpu.SemaphoreType.DMA,
        pltpu.SemaphoreType.DMA,
    ],
)(_agg_body)


def _h1_body(a_hbm, b_hbm, c_hbm, src_hbm, dst_hbm, out_hbm,
             srcv, dstv, asv, bdv, cv, hv, sem0, sem1, sem2):
    c = lax.axis_index("c")
    s = lax.axis_index("s")
    wid = s * NC + c

    @pl.loop(wid, NCHUNK, step=NC * NS)
    def _chunk(t):
        eb = t * K
        pltpu.sync_copy(src_hbm.at[pl.ds(eb, K)], srcv)
        pltpu.sync_copy(dst_hbm.at[pl.ds(eb, K)], dstv)
        ca = pltpu.async_copy(a_hbm.at[srcv], asv, sem0)
        cb = pltpu.async_copy(b_hbm.at[dstv], bdv, sem1)
        cc = pltpu.async_copy(c_hbm.at[pl.ds(eb, K)], cv, sem2)
        ca.wait()
        cb.wait()
        cc.wait()
        _edge_compute(asv, bdv, cv, hv)
        pltpu.sync_copy(hv, out_hbm.at[pl.ds(eb, K)])


_h1_kernel = functools.partial(
    pl.kernel,
    out_type=jax.ShapeDtypeStruct((E, H), jnp.float32),
    mesh=_mesh,
    compiler_params=_sc_params,
    scratch_types=[
        pltpu.VMEM((K,), jnp.int32),
        pltpu.VMEM((K,), jnp.int32),
        pltpu.VMEM((K, D), jnp.float32),
        pltpu.VMEM((K, D), jnp.float32),
        pltpu.VMEM((K, H), jnp.float32),
        pltpu.VMEM((K, H), jnp.float32),
        pltpu.SemaphoreType.DMA,
        pltpu.SemaphoreType.DMA,
        pltpu.SemaphoreType.DMA,
    ],
)(_h1_body)


BN = 2000
BEC = 8000


def _embed_body(nf_ref, we_ref, be_ref, w1a_ref, w1b_ref, x_ref, a_ref, b_ref):
    x = jnp.maximum(
        jnp.dot(nf_ref[...], we_ref[...], preferred_element_type=jnp.float32)
        + be_ref[...], 0.0)
    x_ref[...] = x
    a_ref[...] = jnp.dot(x, w1a_ref[...], preferred_element_type=jnp.float32)
    b_ref[...] = jnp.dot(x, w1b_ref[...], preferred_element_type=jnp.float32)


def _embed_call(nf, we, be2, w1a, w1b):
    n32 = jax.ShapeDtypeStruct((N, D), jnp.float32)
    return pl.pallas_call(
        _embed_body,
        grid=(N // BN,),
        in_specs=[
            pl.BlockSpec((BN, 7), lambda i: (i, 0)),
            pl.BlockSpec((7, D), lambda i: (0, 0)),
            pl.BlockSpec((1, D), lambda i: (0, 0)),
            pl.BlockSpec((D, D), lambda i: (0, 0)),
            pl.BlockSpec((D, D), lambda i: (0, 0)),
        ],
        out_specs=[pl.BlockSpec((BN, D), lambda i: (i, 0))] * 3,
        out_shape=[n32, n32, n32],
    )(nf, we, be2, w1a, w1b)


def _edgec_body(ea_ref, w_ref, b_ref, c0_ref, c1_ref, c2_ref, c3_ref):
    ea = ea_ref[...]
    for k, cref in enumerate((c0_ref, c1_ref, c2_ref, c3_ref)):
        cref[...] = (
            jnp.dot(ea, w_ref[k], preferred_element_type=jnp.float32) + b_ref[k]
        )


def _edgec_call(ea, wstack, bstack):
    e32 = jax.ShapeDtypeStruct((E, H), jnp.float32)
    return pl.pallas_call(
        _edgec_body,
        grid=(E // BEC,),
        in_specs=[
            pl.BlockSpec((BEC, 16), lambda i: (i, 0)),
            pl.BlockSpec((4, 16, H), lambda i: (0, 0, 0)),
            pl.BlockSpec((4, H), lambda i: (0, 0)),
        ],
        out_specs=[pl.BlockSpec((BEC, H), lambda i: (i, 0))] * 4,
        out_shape=[e32] * 4,
    )(ea, wstack, bstack)


def _update_body(x_ref, agg2_ref, cnt2_ref, wm2_ref, bm2_ref, wua_ref, wub_ref,
                 bu_ref, g_ref, be_ref, wna_ref, wnb_ref,
                 xo_ref, ao_ref, bo_ref):
    x = x_ref[...]
    aggs = agg2_ref[0] + agg2_ref[1]
    cnt = cnt2_ref[0] + cnt2_ref[1]
    msum = jnp.dot(aggs, wm2_ref[...], preferred_element_type=jnp.float32) \
        + cnt * bm2_ref[...]
    agg = msum / (cnt + 1e-6)
    upd = jnp.maximum(
        jnp.dot(x, wua_ref[...], preferred_element_type=jnp.float32)
        + jnp.dot(agg, wub_ref[...], preferred_element_type=jnp.float32)
        + bu_ref[...], 0.0)
    xn = x + upd
    mu = jnp.mean(xn, axis=-1, keepdims=True)
    var = jnp.mean((xn - mu) ** 2, axis=-1, keepdims=True)
    xn = (xn - mu) / jnp.sqrt(var + 1e-5) * g_ref[...] + be_ref[...]
    xo_ref[...] = xn
    ao_ref[...] = jnp.dot(xn, wna_ref[...], preferred_element_type=jnp.float32)
    bo_ref[...] = jnp.dot(xn, wnb_ref[...], preferred_element_type=jnp.float32)


def _update_call(x, agg2, cnt2, wm2, bm2_2, wua, wub, bu2, g2, be2, wna, wnb):
    n32 = jax.ShapeDtypeStruct((N, D), jnp.float32)
    wspec = pl.BlockSpec((D, D), lambda i: (0, 0))
    vspec = pl.BlockSpec((1, D), lambda i: (0, 0))
    return pl.pallas_call(
        _update_body,
        grid=(N // BN,),
        in_specs=[
            pl.BlockSpec((BN, D), lambda i: (i, 0)),
            pl.BlockSpec((NC, BN, H), lambda i: (0, i, 0)),
            pl.BlockSpec((NC, BN, 1), lambda i: (0, i, 0)),
            wspec, vspec, wspec, wspec, vspec, vspec, vspec, wspec, wspec,
        ],
        out_specs=[pl.BlockSpec((BN, D), lambda i: (i, 0))] * 3,
        out_shape=[n32, n32, n32],
    )(x, agg2, cnt2, wm2, bm2_2, wua, wub, bu2, g2, be2, wna, wnb)


def _pred_body(h1_ref, wp2_ref, bp2_ref, wp3_ref, bp3_ref, out_ref):
    h2 = jnp.maximum(
        jnp.dot(h1_ref[...], wp2_ref[...], preferred_element_type=jnp.float32)
        + bp2_ref[...], 0.0)
    z = jnp.dot(h2, wp3_ref[...], preferred_element_type=jnp.float32) + bp3_ref[...]
    out_ref[...] = jnp.tanh(z)


def _pred_call(h1, wp2, bp2_2, wp3, bp3_2):
    return pl.pallas_call(
        _pred_body,
        grid=(E // BEC,),
        in_specs=[
            pl.BlockSpec((BEC, H), lambda i: (i, 0)),
            pl.BlockSpec((H, H // 2), lambda i: (0, 0)),
            pl.BlockSpec((1, H // 2), lambda i: (0, 0)),
            pl.BlockSpec((H // 2, 1), lambda i: (0, 0)),
            pl.BlockSpec((1, 1), lambda i: (0, 0)),
        ],
        out_specs=pl.BlockSpec((BEC, 1), lambda i: (i, 0)),
        out_shape=jax.ShapeDtypeStruct((E, 1), jnp.float32),
    )(h1, wp2, bp2_2, wp3, bp3_2)


def kernel(node_features, edge_index, edge_attr, W_embed, b_embed,
           Wm1_0, bm1_0, Wm2_0, bm2_0, Wu_0, bu_0, g_0, be_0,
           Wm1_1, bm1_1, Wm2_1, bm2_1, Wu_1, bu_1, g_1, be_1,
           Wm1_2, bm1_2, Wm2_2, bm2_2, Wu_2, bu_2, g_2, be_2,
           Wp1, bp1, Wp2, bp2, Wp3, bp3):
    src = edge_index[0]
    dst = edge_index[1]

    wm1 = [Wm1_0, Wm1_1, Wm1_2]
    bm1 = [bm1_0, bm1_1, bm1_2]
    wm2 = [Wm2_0, Wm2_1, Wm2_2]
    bm2 = [bm2_0, bm2_1, bm2_2]
    wu = [Wu_0, Wu_1, Wu_2]
    bu = [bu_0, bu_1, bu_2]
    g = [g_0, g_1, g_2]
    be = [be_0, be_1, be_2]

    cnt2 = _count_kernel(dst).reshape(NC, NPAD, 1)

    wstack = jnp.stack([wm1[0][2 * D:], wm1[1][2 * D:], wm1[2][2 * D:],
                        Wp1[2 * D:]])
    bstack = jnp.stack([bm1[0], bm1[1], bm1[2], bp1])
    c_all = _edgec_call(edge_attr, wstack, bstack)

    x, a, b = _embed_call(node_features, W_embed, b_embed[None, :],
                          wm1[0][:D], wm1[0][D:2 * D])

    for l in range(3):
        agg2 = _agg_kernel(a, b, c_all[l], src, dst)
        if l < 2:
            wna, wnb = wm1[l + 1][:D], wm1[l + 1][D:2 * D]
        else:
            wna, wnb = Wp1[:D], Wp1[D:2 * D]
        x, a, b = _update_call(
            x, agg2, cnt2, wm2[l], bm2[l][None, :], wu[l][:D], wu[l][D:],
            bu[l][None, :], g[l][None, :], be[l][None, :], wna, wnb)

    h1 = _h1_kernel(a, b, c_all[3], src, dst)
    p = _pred_call(h1, Wp2, bp2[None, :], Wp3, bp3[None, :])
    return p[:, 0]


# speedup vs baseline: 4.3032x; 1.0171x over previous
import functools

import jax
import jax.numpy as jnp
from jax import lax
from jax.experimental import pallas as pl
from jax.experimental.pallas import tpu as pltpu
from jax.experimental.pallas import tpu_sc as plsc

N = 50000
E = 800000
D = 32
H = 32

NC = 2
NS = 16

KC = 128
NCHUNK = E // KC
NW = NC * NS
K = 112
CPW = 224
NCHUNK_PAD = NW * CPW
E_PAD = NCHUNK_PAD * K
NPAD = 51200
RPS = NPAD // NS
NPAD_A = 50048
RPA = NPAD_A // NS

_mesh = plsc.VectorSubcoreMesh(core_axis_name="c", subcore_axis_name="s")
_sc_params = pltpu.CompilerParams(use_tc_tiling_on_sc=False)


def _cnt_body(dst_hbm, out_hbm, dstv, onesv, zb, cnt_sh, sem):
    c = lax.axis_index("c")
    s = lax.axis_index("s")
    wid = s * NC + c

    @pl.loop(0, 8)
    def _fill_ones(i):
        onesv[pl.ds(i * 16, 16)] = jnp.full((16,), 1.0, jnp.float32)

    @pl.loop(0, 50)
    def _fill_zeros(i):
        zb[pl.ds(i * 16, 16)] = jnp.zeros((16,), jnp.float32)

    base = s * RPS

    @pl.loop(0, RPS // 800)
    def _zero(k):
        pltpu.sync_copy(zb, cnt_sh.at[pl.ds(base + k * 800, 800)])

    plsc.subcore_barrier()

    @pl.loop(wid, NCHUNK, step=NW)
    def _scatter(t):
        eb = t * KC
        pltpu.sync_copy(dst_hbm.at[pl.ds(eb, KC)], dstv)
        pltpu.sync_copy(onesv, cnt_sh.at[dstv], add=True)

    plsc.subcore_barrier()

    @pl.loop(0, RPS // 800)
    def _drain(k):
        o = base + k * 800
        pltpu.sync_copy(cnt_sh.at[pl.ds(o, 800)], zb)
        pltpu.sync_copy(zb, out_hbm.at[pl.ds(c * NPAD + o, 800)])


_count_kernel = functools.partial(
    pl.kernel,
    out_type=jax.ShapeDtypeStruct((NC * NPAD,), jnp.float32),
    mesh=_mesh,
    compiler_params=_sc_params,
    scratch_types=[
        pltpu.VMEM((KC,), jnp.int32),
        pltpu.VMEM((KC,), jnp.float32),
        pltpu.VMEM((800,), jnp.float32),
        pltpu.VMEM_SHARED((NPAD,), jnp.float32),
        pltpu.SemaphoreType.DMA,
    ],
)(_cnt_body)


NSLOT_AGG = 2
NSLOT_H1 = 4


def _compute_h(asv, bdv, cv, hv):
    @pl.loop(0, K, unroll=8)
    def _rows(i):
        for j in (0, 16):
            hv[i, pl.ds(j, 16)] = jnp.maximum(
                asv[i, pl.ds(j, 16)] + bdv[i, pl.ds(j, 16)] + cv[i, pl.ds(j, 16)],
                0.0,
            )


def _snapshot_dst(iv, dsc):
    @pl.loop(0, K // 16)
    def _cp(q):
        dsc[pl.ds(q * 16, 16)] = iv[1, pl.ds(q * 16, 16)]


def _start_gathers(a_hbm, b_hbm, c_hbm, iv, asv, bdv, cv, gsem, eb):
    pltpu.async_copy(a_hbm.at[iv.at[0]], asv, gsem)
    pltpu.async_copy(b_hbm.at[iv.at[1]], bdv, gsem)
    pltpu.async_copy(c_hbm.at[pl.ds(eb, K)], cv, gsem)


def _wait_gathers(a_hbm, b_hbm, c_hbm, iv, asv, bdv, cv, gsem, eb):
    pltpu.make_async_copy(a_hbm.at[iv.at[0]], asv, gsem).wait()
    pltpu.make_async_copy(b_hbm.at[iv.at[1]], bdv, gsem).wait()
    pltpu.make_async_copy(c_hbm.at[pl.ds(eb, K)], cv, gsem).wait()


def _slot_scratch(nslot):
    return [
        pltpu.VMEM((nslot, 2, K), jnp.int32),
        [pltpu.VMEM((K,), jnp.int32) for _ in range(nslot)],
        [pltpu.VMEM((K, D), jnp.float32) for _ in range(nslot)],
        [pltpu.VMEM((K, D), jnp.float32) for _ in range(nslot)],
        [pltpu.VMEM((K, H), jnp.float32) for _ in range(nslot)],
        [pltpu.VMEM((K, H), jnp.float32) for _ in range(nslot)],
        [pltpu.SemaphoreType.DMA for _ in range(nslot)],
        [pltpu.SemaphoreType.DMA for _ in range(nslot)],
    ]


def _agg_body(a_hbm, b_hbm, c_hbm, sd_hbm, out_hbm,
              idxv, dsc, asv, bdv, cv, hv, gsem, ssem, agg_sh):
    c = lax.axis_index("c")
    s = lax.axis_index("s")
    wid = s * NC + c
    nslot = NSLOT_AGG
    niter = CPW // nslot

    @pl.loop(0, K * 2)
    def _fill_zeros(q):
        hv[0][q // 2, pl.ds((q % 2) * 16, 16)] = jnp.zeros((16,), jnp.float32)

    rbase = s * RPA

    @pl.loop(0, 27)
    def _zero(k):
        pltpu.sync_copy(hv[0], agg_sh.at[pl.ds(rbase + k * K, K)])

    pltpu.sync_copy(hv[0].at[pl.ds(0, 104)],
                    agg_sh.at[pl.ds(rbase + 27 * K, 104)])

    plsc.subcore_barrier()

    for i in range(nslot):
        t = wid + NW * i
        pltpu.sync_copy(sd_hbm.at[t], idxv.at[i])
        _start_gathers(a_hbm, b_hbm, c_hbm, idxv.at[i], asv[i], bdv[i],
                       cv[i], gsem[i], t * K)

    @pl.loop(0, niter)
    def _iter(j):
        for i in range(nslot):
            t = wid + NW * (nslot * j + i)
            eb = t * K
            _wait_gathers(a_hbm, b_hbm, c_hbm, idxv.at[i], asv[i], bdv[i],
                          cv[i], gsem[i], eb)

            @pl.when(j > 0)
            def _wait_prev_scatter():
                pltpu.make_async_copy(hv[i], agg_sh.at[dsc[i]], ssem[i]).wait()

            _compute_h(asv[i], bdv[i], cv[i], hv[i])
            _snapshot_dst(idxv.at[i], dsc[i])
            pltpu.async_copy(hv[i], agg_sh.at[dsc[i]], ssem[i], add=True)

            @pl.when(j < niter - 1)
            def _prefetch():
                t2 = wid + NW * (nslot * (j + 1) + i)
                pltpu.sync_copy(sd_hbm.at[t2], idxv.at[i])
                _start_gathers(a_hbm, b_hbm, c_hbm, idxv.at[i], asv[i],
                               bdv[i], cv[i], gsem[i], t2 * K)

    for i in range(nslot):
        pltpu.make_async_copy(hv[i], agg_sh.at[dsc[i]], ssem[i]).wait()

    plsc.subcore_barrier()

    @pl.loop(0, 27)
    def _drain(k):
        o = rbase + k * K
        pltpu.sync_copy(agg_sh.at[pl.ds(o, K)], hv[0])
        pltpu.sync_copy(hv[0], out_hbm.at[c, pl.ds(o, K)])

    o = rbase + 27 * K
    pltpu.sync_copy(agg_sh.at[pl.ds(o, 104)], hv[0].at[pl.ds(0, 104)])
    pltpu.sync_copy(hv[0].at[pl.ds(0, 104)], out_hbm.at[c, pl.ds(o, 104)])


_agg_kernel = functools.partial(
    pl.kernel,
    out_type=jax.ShapeDtypeStruct((NC, NPAD_A, H), jnp.float32),
    mesh=_mesh,
    compiler_params=_sc_params,
    scratch_types=_slot_scratch(NSLOT_AGG) + [
        pltpu.VMEM_SHARED((NPAD_A, H), jnp.float32),
    ],
)(_agg_body)


def _h1_body(a_hbm, b_hbm, c_hbm, sd_hbm, out_hbm,
             idxv, dsc, asv, bdv, cv, hv, gsem, ssem):
    c = lax.axis_index("c")
    s = lax.axis_index("s")
    wid = s * NC + c
    nslot = NSLOT_H1
    niter = CPW // nslot

    for i in range(nslot):
        t = wid + NW * i
        pltpu.sync_copy(sd_hbm.at[t], idxv.at[i])
        _start_gathers(a_hbm, b_hbm, c_hbm, idxv.at[i], asv[i], bdv[i],
                       cv[i], gsem[i], t * K)

    @pl.loop(0, niter)
    def _iter(j):
        for i in range(nslot):
            t = wid + NW * (nslot * j + i)
            eb = t * K
            _wait_gathers(a_hbm, b_hbm, c_hbm, idxv.at[i], asv[i], bdv[i],
                          cv[i], gsem[i], eb)

            @pl.when(j > 0)
            def _wait_prev_store():
                pltpu.make_async_copy(hv[i], out_hbm.at[pl.ds(eb, K)],
                                      ssem[i]).wait()

            _compute_h(asv[i], bdv[i], cv[i], hv[i])
            pltpu.async_copy(hv[i], out_hbm.at[pl.ds(eb, K)], ssem[i])

            @pl.when(j < niter - 1)
            def _prefetch():
                t2 = wid + NW * (nslot * (j + 1) + i)
                pltpu.sync_copy(sd_hbm.at[t2], idxv.at[i])
                _start_gathers(a_hbm, b_hbm, c_hbm, idxv.at[i], asv[i],
                               bdv[i], cv[i], gsem[i], t2 * K)

    for i in range(nslot):
        pltpu.make_async_copy(hv[i], out_hbm.at[pl.ds(0, K)], ssem[i]).wait()


_h1_kernel = functools.partial(
    pl.kernel,
    out_type=jax.ShapeDtypeStruct((E_PAD, H), jnp.float32),
    mesh=_mesh,
    compiler_params=_sc_params,
    scratch_types=_slot_scratch(NSLOT_H1),
)(_h1_body)


BN = 2000
BEC = 7168


def _embed_body(nf_ref, we_ref, be_ref, w1a_ref, w1b_ref, x_ref, a_ref, b_ref):
    x = jnp.maximum(
        jnp.dot(nf_ref[...], we_ref[...], preferred_element_type=jnp.float32)
        + be_ref[...], 0.0)
    x_ref[...] = x
    a_ref[...] = jnp.dot(x, w1a_ref[...], preferred_element_type=jnp.float32)
    b_ref[...] = jnp.dot(x, w1b_ref[...], preferred_element_type=jnp.float32)


def _embed_call(nf, we, be2, w1a, w1b):
    n32 = jax.ShapeDtypeStruct((N, D), jnp.float32)
    return pl.pallas_call(
        _embed_body,
        grid=(N // BN,),
        in_specs=[
            pl.BlockSpec((BN, 7), lambda i: (i, 0)),
            pl.BlockSpec((7, D), lambda i: (0, 0)),
            pl.BlockSpec((1, D), lambda i: (0, 0)),
            pl.BlockSpec((D, D), lambda i: (0, 0)),
            pl.BlockSpec((D, D), lambda i: (0, 0)),
        ],
        out_specs=[pl.BlockSpec((BN, D), lambda i: (i, 0))] * 3,
        out_shape=[n32, n32, n32],
    )(nf, we, be2, w1a, w1b)


def _edgec_body(ea_ref, w_ref, b_ref, c0_ref, c1_ref, c2_ref, c3_ref):
    ea = ea_ref[...]
    rows = pl.program_id(0) * BEC + lax.broadcasted_iota(
        jnp.int32, (BEC, 1), 0)
    valid = rows < E
    for k, cref in enumerate((c0_ref, c1_ref, c2_ref, c3_ref)):
        v = jnp.dot(ea, w_ref[k], preferred_element_type=jnp.float32) + b_ref[k]
        cref[...] = jnp.where(valid, v, -1e30)


def _edgec_call(ea, wstack, bstack):
    e32 = jax.ShapeDtypeStruct((E_PAD, H), jnp.float32)
    return pl.pallas_call(
        _edgec_body,
        grid=(E_PAD // BEC,),
        in_specs=[
            pl.BlockSpec((BEC, 16), lambda i: (i, 0)),
            pl.BlockSpec((4, 16, H), lambda i: (0, 0, 0)),
            pl.BlockSpec((4, H), lambda i: (0, 0)),
        ],
        out_specs=[pl.BlockSpec((BEC, H), lambda i: (i, 0))] * 4,
        out_shape=[e32] * 4,
    )(ea, wstack, bstack)


def _update_body(x_ref, agg2_ref, cnt2_ref, wm2_ref, bm2_ref, wua_ref, wub_ref,
                 bu_ref, g_ref, be_ref, wna_ref, wnb_ref,
                 xo_ref, ao_ref, bo_ref):
    x = x_ref[...]
    aggs = agg2_ref[0] + agg2_ref[1]
    cnt = cnt2_ref[0] + cnt2_ref[1]
    msum = jnp.dot(aggs, wm2_ref[...], preferred_element_type=jnp.float32) \
        + cnt * bm2_ref[...]
    agg = msum / (cnt + 1e-6)
    upd = jnp.maximum(
        jnp.dot(x, wua_ref[...], preferred_element_type=jnp.float32)
        + jnp.dot(agg, wub_ref[...], preferred_element_type=jnp.float32)
        + bu_ref[...], 0.0)
    xn = x + upd
    mu = jnp.mean(xn, axis=-1, keepdims=True)
    var = jnp.mean((xn - mu) ** 2, axis=-1, keepdims=True)
    xn = (xn - mu) / jnp.sqrt(var + 1e-5) * g_ref[...] + be_ref[...]
    xo_ref[...] = xn
    ao_ref[...] = jnp.dot(xn, wna_ref[...], preferred_element_type=jnp.float32)
    bo_ref[...] = jnp.dot(xn, wnb_ref[...], preferred_element_type=jnp.float32)


def _update_call(x, agg2, cnt2, wm2, bm2_2, wua, wub, bu2, g2, be2, wna, wnb):
    n32 = jax.ShapeDtypeStruct((N, D), jnp.float32)
    wspec = pl.BlockSpec((D, D), lambda i: (0, 0))
    vspec = pl.BlockSpec((1, D), lambda i: (0, 0))
    return pl.pallas_call(
        _update_body,
        grid=(N // BN,),
        in_specs=[
            pl.BlockSpec((BN, D), lambda i: (i, 0)),
            pl.BlockSpec((NC, BN, H), lambda i: (0, i, 0)),
            pl.BlockSpec((NC, BN, 1), lambda i: (0, i, 0)),
            wspec, vspec, wspec, wspec, vspec, vspec, vspec, wspec, wspec,
        ],
        out_specs=[pl.BlockSpec((BN, D), lambda i: (i, 0))] * 3,
        out_shape=[n32, n32, n32],
    )(x, agg2, cnt2, wm2, bm2_2, wua, wub, bu2, g2, be2, wna, wnb)


def _pred_body(h1_ref, wp2_ref, bp2_ref, wp3_ref, bp3_ref, out_ref):
    h2 = jnp.maximum(
        jnp.dot(h1_ref[...], wp2_ref[...], preferred_element_type=jnp.float32)
        + bp2_ref[...], 0.0)
    z = jnp.dot(h2, wp3_ref[...], preferred_element_type=jnp.float32) + bp3_ref[...]
    out_ref[...] = jnp.tanh(z)


def _pred_call(h1, wp2, bp2_2, wp3, bp3_2):
    return pl.pallas_call(
        _pred_body,
        grid=(E_PAD // BEC,),
        in_specs=[
            pl.BlockSpec((BEC, H), lambda i: (i, 0)),
            pl.BlockSpec((H, H // 2), lambda i: (0, 0)),
            pl.BlockSpec((1, H // 2), lambda i: (0, 0)),
            pl.BlockSpec((H // 2, 1), lambda i: (0, 0)),
            pl.BlockSpec((1, 1), lambda i: (0, 0)),
        ],
        out_specs=pl.BlockSpec((BEC, 1), lambda i: (i, 0)),
        out_shape=jax.ShapeDtypeStruct((E_PAD, 1), jnp.float32),
    )(h1, wp2, bp2_2, wp3, bp3_2)


def kernel(node_features, edge_index, edge_attr, W_embed, b_embed,
           Wm1_0, bm1_0, Wm2_0, bm2_0, Wu_0, bu_0, g_0, be_0,
           Wm1_1, bm1_1, Wm2_1, bm2_1, Wu_1, bu_1, g_1, be_1,
           Wm1_2, bm1_2, Wm2_2, bm2_2, Wu_2, bu_2, g_2, be_2,
           Wp1, bp1, Wp2, bp2, Wp3, bp3):
    src = edge_index[0]
    dst = edge_index[1]

    pad = E_PAD - E
    src_p = jnp.concatenate([src, jnp.zeros((pad,), jnp.int32)])
    dst_p = jnp.concatenate([dst, jnp.zeros((pad,), jnp.int32)])
    sd = jnp.stack([src_p.reshape(NCHUNK_PAD, K),
                    dst_p.reshape(NCHUNK_PAD, K)], axis=1)

    wm1 = [Wm1_0, Wm1_1, Wm1_2]
    bm1 = [bm1_0, bm1_1, bm1_2]
    wm2 = [Wm2_0, Wm2_1, Wm2_2]
    bm2 = [bm2_0, bm2_1, bm2_2]
    wu = [Wu_0, Wu_1, Wu_2]
    bu = [bu_0, bu_1, bu_2]
    g = [g_0, g_1, g_2]
    be = [be_0, be_1, be_2]

    cnt2 = _count_kernel(dst).reshape(NC, NPAD, 1)

    wstack = jnp.stack([wm1[0][2 * D:], wm1[1][2 * D:], wm1[2][2 * D:],
                        Wp1[2 * D:]])
    bstack = jnp.stack([bm1[0], bm1[1], bm1[2], bp1])
    c_all = _edgec_call(edge_attr, wstack, bstack)

    x, a, b = _embed_call(node_features, W_embed, b_embed[None, :],
                          wm1[0][:D], wm1[0][D:2 * D])

    for l in range(3):
        agg2 = _agg_kernel(a, b, c_all[l], sd)
        if l < 2:
            wna, wnb = wm1[l + 1][:D], wm1[l + 1][D:2 * D]
        else:
            wna, wnb = Wp1[:D], Wp1[D:2 * D]
        x, a, b = _update_call(
            x, agg2, cnt2, wm2[l], bm2[l][None, :], wu[l][:D], wu[l][D:],
            bu[l][None, :], g[l][None, :], be[l][None, :], wna, wnb)

    h1 = _h1_kernel(a, b, c_all[3], sd)
    p = _pred_call(h1, Wp2, bp2[None, :], Wp3, bp3[None, :])
    return p[:E, 0]


# speedup vs baseline: 5.0541x; 1.1745x over previous
import functools

import jax
import jax.numpy as jnp
from jax import lax
from jax.experimental import pallas as pl
from jax.experimental.pallas import tpu as pltpu
from jax.experimental.pallas import tpu_sc as plsc

N = 50000
E = 800000
D = 32
H = 32

NC = 2
NS = 16

KC = 128
NCHUNK = E // KC
NW = NC * NS
K = 112
CPW = 224
NCHUNK_PAD = NW * CPW
E_PAD = NCHUNK_PAD * K
NPAD = 51200
RPS = NPAD // NS
NPAD_A = 50048
RPA = NPAD_A // NS
M4 = E_PAD // 4
BE4 = 1792
GE = E_PAD // (4 * BE4)

_mesh = plsc.VectorSubcoreMesh(core_axis_name="c", subcore_axis_name="s")
_sc_params = pltpu.CompilerParams(use_tc_tiling_on_sc=False)


def _cnt_body(dst_hbm, out_hbm, dstv, onesv, zb, cnt_sh, sem):
    c = lax.axis_index("c")
    s = lax.axis_index("s")
    wid = s * NC + c

    @pl.loop(0, 8)
    def _fill_ones(i):
        onesv[pl.ds(i * 16, 16)] = jnp.full((16,), 1.0, jnp.float32)

    @pl.loop(0, 50)
    def _fill_zeros(i):
        zb[pl.ds(i * 16, 16)] = jnp.zeros((16,), jnp.float32)

    base = s * RPS

    @pl.loop(0, RPS // 800)
    def _zero(k):
        pltpu.sync_copy(zb, cnt_sh.at[pl.ds(base + k * 800, 800)])

    plsc.subcore_barrier()

    @pl.loop(wid, NCHUNK, step=NW)
    def _scatter(t):
        eb = t * KC
        pltpu.sync_copy(dst_hbm.at[pl.ds(eb, KC)], dstv)
        pltpu.sync_copy(onesv, cnt_sh.at[dstv], add=True)

    plsc.subcore_barrier()

    @pl.loop(0, RPS // 800)
    def _drain(k):
        o = base + k * 800
        pltpu.sync_copy(cnt_sh.at[pl.ds(o, 800)], zb)
        pltpu.sync_copy(zb, out_hbm.at[pl.ds(c * NPAD + o, 800)])


_count_kernel = functools.partial(
    pl.kernel,
    out_type=jax.ShapeDtypeStruct((NC * NPAD,), jnp.float32),
    mesh=_mesh,
    compiler_params=_sc_params,
    scratch_types=[
        pltpu.VMEM((KC,), jnp.int32),
        pltpu.VMEM((KC,), jnp.float32),
        pltpu.VMEM((800,), jnp.float32),
        pltpu.VMEM_SHARED((NPAD,), jnp.float32),
        pltpu.SemaphoreType.DMA,
    ],
)(_cnt_body)


NSLOT_AGG = 2
NSLOT_H1 = 4


def _compute_h(asv, bdv, cv, hv, packed_out):
    @pl.loop(0, K, unroll=8)
    def _rows(i):
        r = i // 4
        cb = (i % 4) * 32
        for j in (0, 16):
            v = jnp.maximum(
                asv[i, pl.ds(j, 16)] + bdv[i, pl.ds(j, 16)]
                + cv[r, pl.ds(cb + j, 16)], 0.0)
            if packed_out:
                hv[r, pl.ds(cb + j, 16)] = v
            else:
                hv[i, pl.ds(j, 16)] = v


def _snapshot_dst(iv, dsc):
    @pl.loop(0, K // 16)
    def _cp(q):
        dsc[pl.ds(q * 16, 16)] = iv[1, pl.ds(q * 16, 16)]


def _c4_slice(c_hbm, t):
    return c_hbm.at[pl.ds((t // 64) * BE4 + (t % 64) * 28, 28)]


def _start_gathers(a_hbm, b_hbm, c_hbm, iv, asv, bdv, cv, gsem, t):
    pltpu.async_copy(a_hbm.at[iv.at[0]], asv, gsem)
    pltpu.async_copy(b_hbm.at[iv.at[1]], bdv, gsem)
    pltpu.async_copy(_c4_slice(c_hbm, t), cv, gsem)


def _wait_gathers(a_hbm, b_hbm, c_hbm, iv, asv, bdv, cv, gsem, t):
    pltpu.make_async_copy(a_hbm.at[iv.at[0]], asv, gsem).wait()
    pltpu.make_async_copy(b_hbm.at[iv.at[1]], bdv, gsem).wait()
    pltpu.make_async_copy(_c4_slice(c_hbm, t), cv, gsem).wait()


def _slot_scratch(nslot, packed_hv):
    hv_shape = (28, 128) if packed_hv else (K, H)
    return [
        pltpu.VMEM((nslot, 2, K), jnp.int32),
        [pltpu.VMEM((K,), jnp.int32) for _ in range(nslot)],
        [pltpu.VMEM((K, D), jnp.float32) for _ in range(nslot)],
        [pltpu.VMEM((K, D), jnp.float32) for _ in range(nslot)],
        [pltpu.VMEM((28, 128), jnp.float32) for _ in range(nslot)],
        [pltpu.VMEM(hv_shape, jnp.float32) for _ in range(nslot)],
        [pltpu.SemaphoreType.DMA for _ in range(nslot)],
        [pltpu.SemaphoreType.DMA for _ in range(nslot)],
    ]


def _agg_body(a_hbm, b_hbm, c_hbm, sd_hbm, out_hbm,
              idxv, dsc, asv, bdv, cv, hv, gsem, ssem, agg_sh):
    c = lax.axis_index("c")
    s = lax.axis_index("s")
    wid = s * NC + c
    nslot = NSLOT_AGG
    niter = CPW // nslot

    @pl.loop(0, K * 2)
    def _fill_zeros(q):
        hv[0][q // 2, pl.ds((q % 2) * 16, 16)] = jnp.zeros((16,), jnp.float32)

    rbase = s * RPA

    @pl.loop(0, 27)
    def _zero(k):
        pltpu.sync_copy(hv[0], agg_sh.at[pl.ds(rbase + k * K, K)])

    pltpu.sync_copy(hv[0].at[pl.ds(0, 104)],
                    agg_sh.at[pl.ds(rbase + 27 * K, 104)])

    plsc.subcore_barrier()

    for i in range(nslot):
        t = wid + NW * i
        pltpu.sync_copy(sd_hbm.at[t], idxv.at[i])
        _start_gathers(a_hbm, b_hbm, c_hbm, idxv.at[i], asv[i], bdv[i],
                       cv[i], gsem[i], t)

    @pl.loop(0, niter)
    def _iter(j):
        for i in range(nslot):
            t = wid + NW * (nslot * j + i)
            _wait_gathers(a_hbm, b_hbm, c_hbm, idxv.at[i], asv[i], bdv[i],
                          cv[i], gsem[i], t)

            @pl.when(j > 0)
            def _wait_prev_scatter():
                pltpu.make_async_copy(hv[i], agg_sh.at[dsc[i]], ssem[i]).wait()

            _compute_h(asv[i], bdv[i], cv[i], hv[i], False)
            _snapshot_dst(idxv.at[i], dsc[i])
            pltpu.async_copy(hv[i], agg_sh.at[dsc[i]], ssem[i], add=True)

            @pl.when(j < niter - 1)
            def _prefetch():
                t2 = wid + NW * (nslot * (j + 1) + i)
                pltpu.sync_copy(sd_hbm.at[t2], idxv.at[i])
                _start_gathers(a_hbm, b_hbm, c_hbm, idxv.at[i], asv[i],
                               bdv[i], cv[i], gsem[i], t2)

    for i in range(nslot):
        pltpu.make_async_copy(hv[i], agg_sh.at[dsc[i]], ssem[i]).wait()

    plsc.subcore_barrier()

    @pl.loop(0, 27)
    def _drain(k):
        o = rbase + k * K
        pltpu.sync_copy(agg_sh.at[pl.ds(o, K)], hv[0])
        pltpu.sync_copy(hv[0], out_hbm.at[c, pl.ds(o, K)])

    o = rbase + 27 * K
    pltpu.sync_copy(agg_sh.at[pl.ds(o, 104)], hv[0].at[pl.ds(0, 104)])
    pltpu.sync_copy(hv[0].at[pl.ds(0, 104)], out_hbm.at[c, pl.ds(o, 104)])


_agg_kernel = functools.partial(
    pl.kernel,
    out_type=jax.ShapeDtypeStruct((NC, NPAD_A, H), jnp.float32),
    mesh=_mesh,
    compiler_params=_sc_params,
    scratch_types=_slot_scratch(NSLOT_AGG, False) + [
        pltpu.VMEM_SHARED((NPAD_A, H), jnp.float32),
    ],
)(_agg_body)


def _h1_body(a_hbm, b_hbm, c_hbm, sd_hbm, out_hbm,
             idxv, dsc, asv, bdv, cv, hv, gsem, ssem):
    c = lax.axis_index("c")
    s = lax.axis_index("s")
    wid = s * NC + c
    nslot = NSLOT_H1
    niter = CPW // nslot

    for i in range(nslot):
        t = wid + NW * i
        pltpu.sync_copy(sd_hbm.at[t], idxv.at[i])
        _start_gathers(a_hbm, b_hbm, c_hbm, idxv.at[i], asv[i], bdv[i],
                       cv[i], gsem[i], t)

    @pl.loop(0, niter)
    def _iter(j):
        for i in range(nslot):
            t = wid + NW * (nslot * j + i)
            _wait_gathers(a_hbm, b_hbm, c_hbm, idxv.at[i], asv[i], bdv[i],
                          cv[i], gsem[i], t)

            @pl.when(j > 0)
            def _wait_prev_store():
                pltpu.make_async_copy(hv[i], _c4_slice(out_hbm, t),
                                      ssem[i]).wait()

            _compute_h(asv[i], bdv[i], cv[i], hv[i], True)
            pltpu.async_copy(hv[i], _c4_slice(out_hbm, t), ssem[i])

            @pl.when(j < niter - 1)
            def _prefetch():
                t2 = wid + NW * (nslot * (j + 1) + i)
                pltpu.sync_copy(sd_hbm.at[t2], idxv.at[i])
                _start_gathers(a_hbm, b_hbm, c_hbm, idxv.at[i], asv[i],
                               bdv[i], cv[i], gsem[i], t2)

    for i in range(nslot):
        pltpu.make_async_copy(hv[i], _c4_slice(out_hbm, 0), ssem[i]).wait()


_h1_kernel = functools.partial(
    pl.kernel,
    out_type=jax.ShapeDtypeStruct((M4, 128), jnp.float32),
    mesh=_mesh,
    compiler_params=_sc_params,
    scratch_types=_slot_scratch(NSLOT_H1, True),
)(_h1_body)


BN = 2000
BEC = 7168


def _embed_body(nf_ref, we_ref, be_ref, w1a_ref, w1b_ref, x_ref, a_ref, b_ref):
    x = jnp.maximum(
        jnp.dot(nf_ref[...], we_ref[...], preferred_element_type=jnp.float32)
        + be_ref[...], 0.0)
    x_ref[...] = x
    a_ref[...] = jnp.dot(x, w1a_ref[...], preferred_element_type=jnp.float32)
    b_ref[...] = jnp.dot(x, w1b_ref[...], preferred_element_type=jnp.float32)


def _embed_call(nf, we, be2, w1a, w1b):
    n32 = jax.ShapeDtypeStruct((N, D), jnp.float32)
    return pl.pallas_call(
        _embed_body,
        grid=(N // BN,),
        in_specs=[
            pl.BlockSpec((BN, 7), lambda i: (i, 0)),
            pl.BlockSpec((7, D), lambda i: (0, 0)),
            pl.BlockSpec((1, D), lambda i: (0, 0)),
            pl.BlockSpec((D, D), lambda i: (0, 0)),
            pl.BlockSpec((D, D), lambda i: (0, 0)),
        ],
        out_specs=[pl.BlockSpec((BN, D), lambda i: (i, 0))] * 3,
        out_shape=[n32, n32, n32],
    )(nf, we, be2, w1a, w1b)


def _edgec_body(ea0_ref, ea1_ref, ea2_ref, ea3_ref, w_ref, b_ref,
                c0_ref, c1_ref, c2_ref, c3_ref):
    eas = (ea0_ref[...], ea1_ref[...], ea2_ref[...], ea3_ref[...])
    i = pl.program_id(0)
    iota = lax.broadcasted_iota(jnp.int32, (BE4, 1), 0)
    valids = [(4 * i + j) * BE4 + iota < E for j in range(4)]
    for l, cref in enumerate((c0_ref, c1_ref, c2_ref, c3_ref)):
        parts = []
        for j in range(4):
            v = jnp.dot(eas[j], w_ref[l],
                        preferred_element_type=jnp.float32) + b_ref[l]
            parts.append(jnp.where(valids[j], v, -1e30))
        cref[...] = jnp.concatenate(parts, axis=1)


def _edgec_call(ea, wstack, bstack):
    e32 = jax.ShapeDtypeStruct((M4, 128), jnp.float32)

    last = (E - 1) // BE4

    def easpec(j):
        return pl.BlockSpec(
            (BE4, 16), lambda i, j=j: (jnp.minimum(4 * i + j, last), 0))

    return pl.pallas_call(
        _edgec_body,
        grid=(GE,),
        in_specs=[
            easpec(0), easpec(1), easpec(2), easpec(3),
            pl.BlockSpec((4, 16, H), lambda i: (0, 0, 0)),
            pl.BlockSpec((4, H), lambda i: (0, 0)),
        ],
        out_specs=[pl.BlockSpec((BE4, 128), lambda i: (i, 0))] * 4,
        out_shape=[e32] * 4,
    )(ea, ea, ea, ea, wstack, bstack)


def _update_body(x_ref, agg2_ref, cnt2_ref, wm2_ref, bm2_ref, wua_ref, wub_ref,
                 bu_ref, g_ref, be_ref, wna_ref, wnb_ref,
                 xo_ref, ao_ref, bo_ref):
    x = x_ref[...]
    aggs = agg2_ref[0] + agg2_ref[1]
    cnt = cnt2_ref[0] + cnt2_ref[1]
    msum = jnp.dot(aggs, wm2_ref[...], preferred_element_type=jnp.float32) \
        + cnt * bm2_ref[...]
    agg = msum / (cnt + 1e-6)
    upd = jnp.maximum(
        jnp.dot(x, wua_ref[...], preferred_element_type=jnp.float32)
        + jnp.dot(agg, wub_ref[...], preferred_element_type=jnp.float32)
        + bu_ref[...], 0.0)
    xn = x + upd
    mu = jnp.mean(xn, axis=-1, keepdims=True)
    var = jnp.mean((xn - mu) ** 2, axis=-1, keepdims=True)
    xn = (xn - mu) / jnp.sqrt(var + 1e-5) * g_ref[...] + be_ref[...]
    xo_ref[...] = xn
    ao_ref[...] = jnp.dot(xn, wna_ref[...], preferred_element_type=jnp.float32)
    bo_ref[...] = jnp.dot(xn, wnb_ref[...], preferred_element_type=jnp.float32)


def _update_call(x, agg2, cnt2, wm2, bm2_2, wua, wub, bu2, g2, be2, wna, wnb):
    n32 = jax.ShapeDtypeStruct((N, D), jnp.float32)
    wspec = pl.BlockSpec((D, D), lambda i: (0, 0))
    vspec = pl.BlockSpec((1, D), lambda i: (0, 0))
    return pl.pallas_call(
        _update_body,
        grid=(N // BN,),
        in_specs=[
            pl.BlockSpec((BN, D), lambda i: (i, 0)),
            pl.BlockSpec((NC, BN, H), lambda i: (0, i, 0)),
            pl.BlockSpec((NC, BN, 1), lambda i: (0, i, 0)),
            wspec, vspec, wspec, wspec, vspec, vspec, vspec, wspec, wspec,
        ],
        out_specs=[pl.BlockSpec((BN, D), lambda i: (i, 0))] * 3,
        out_shape=[n32, n32, n32],
    )(x, agg2, cnt2, wm2, bm2_2, wua, wub, bu2, g2, be2, wna, wnb)


def _pred_body(h4_ref, wp2_ref, bp2_ref, wp3_ref, bp3_ref, out_ref):
    h4 = h4_ref[...]
    parts = []
    for j in range(4):
        h2 = jnp.maximum(
            jnp.dot(h4[:, 32 * j:32 * (j + 1)], wp2_ref[...],
                    preferred_element_type=jnp.float32) + bp2_ref[...], 0.0)
        z = jnp.dot(h2, wp3_ref[...],
                    preferred_element_type=jnp.float32) + bp3_ref[...]
        parts.append(jnp.tanh(z))
    out_ref[...] = jnp.concatenate(parts, axis=1)


def _pred_call(h4, wp2, bp2_2, wp3, bp3_2):
    return pl.pallas_call(
        _pred_body,
        grid=(GE * 4,),
        in_specs=[
            pl.BlockSpec((BE4 // 4, 128), lambda i: (i, 0)),
            pl.BlockSpec((H, H // 2), lambda i: (0, 0)),
            pl.BlockSpec((1, H // 2), lambda i: (0, 0)),
            pl.BlockSpec((H // 2, 1), lambda i: (0, 0)),
            pl.BlockSpec((1, 1), lambda i: (0, 0)),
        ],
        out_specs=pl.BlockSpec((BE4 // 4, 4), lambda i: (i, 0)),
        out_shape=jax.ShapeDtypeStruct((M4, 4), jnp.float32),
    )(h4, wp2, bp2_2, wp3, bp3_2)


def kernel(node_features, edge_index, edge_attr, W_embed, b_embed,
           Wm1_0, bm1_0, Wm2_0, bm2_0, Wu_0, bu_0, g_0, be_0,
           Wm1_1, bm1_1, Wm2_1, bm2_1, Wu_1, bu_1, g_1, be_1,
           Wm1_2, bm1_2, Wm2_2, bm2_2, Wu_2, bu_2, g_2, be_2,
           Wp1, bp1, Wp2, bp2, Wp3, bp3):
    src = edge_index[0]
    dst = edge_index[1]

    pad = E_PAD - E
    src_p = jnp.concatenate([src, jnp.zeros((pad,), jnp.int32)])
    dst_p = jnp.concatenate([dst, jnp.zeros((pad,), jnp.int32)])
    def _chunk_perm(v):
        return v.reshape(GE, 4, 64, 28).transpose(0, 2, 3, 1).reshape(
            NCHUNK_PAD, K)

    sd = jnp.stack([_chunk_perm(src_p), _chunk_perm(dst_p)], axis=1)

    wm1 = [Wm1_0, Wm1_1, Wm1_2]
    bm1 = [bm1_0, bm1_1, bm1_2]
    wm2 = [Wm2_0, Wm2_1, Wm2_2]
    bm2 = [bm2_0, bm2_1, bm2_2]
    wu = [Wu_0, Wu_1, Wu_2]
    bu = [bu_0, bu_1, bu_2]
    g = [g_0, g_1, g_2]
    be = [be_0, be_1, be_2]

    cnt2 = _count_kernel(dst).reshape(NC, NPAD, 1)

    wstack = jnp.stack([wm1[0][2 * D:], wm1[1][2 * D:], wm1[2][2 * D:],
                        Wp1[2 * D:]])
    bstack = jnp.stack([bm1[0], bm1[1], bm1[2], bp1])
    c_all = _edgec_call(edge_attr, wstack, bstack)

    x, a, b = _embed_call(node_features, W_embed, b_embed[None, :],
                          wm1[0][:D], wm1[0][D:2 * D])

    for l in range(3):
        agg2 = _agg_kernel(a, b, c_all[l], sd)
        if l < 2:
            wna, wnb = wm1[l + 1][:D], wm1[l + 1][D:2 * D]
        else:
            wna, wnb = Wp1[:D], Wp1[D:2 * D]
        x, a, b = _update_call(
            x, agg2, cnt2, wm2[l], bm2[l][None, :], wu[l][:D], wu[l][D:],
            bu[l][None, :], g[l][None, :], be[l][None, :], wna, wnb)

    h4 = _h1_kernel(a, b, c_all[3], sd)
    p4 = _pred_call(h4, Wp2, bp2[None, :], Wp3, bp3[None, :])
    p = p4.reshape(GE, BE4, 4).transpose(0, 2, 1).reshape(E_PAD)
    return p[:E]


# speedup vs baseline: 5.6993x; 1.1277x over previous
import functools

import jax
import jax.numpy as jnp
from jax import lax
from jax.experimental import pallas as pl
from jax.experimental.pallas import tpu as pltpu
from jax.experimental.pallas import tpu_sc as plsc

N = 50000
E = 800000
D = 32
H = 32

NC = 2
NS = 16

KC = 128
NCHUNK = E // KC
NW = NC * NS
K = 112
CPW = 224
NCHUNK_PAD = NW * CPW
E_PAD = NCHUNK_PAD * K
NPAD = 51200
RPS = NPAD // NS
NPAD_A = 50048
RPA = NPAD_A // NS
M4 = E_PAD // 4
BE4 = 1792
GE = E_PAD // (4 * BE4)

_mesh = plsc.VectorSubcoreMesh(core_axis_name="c", subcore_axis_name="s")
_sc_params = pltpu.CompilerParams(use_tc_tiling_on_sc=False)


def _cnt_body(dst_hbm, out_hbm, dstv, onesv, zb, cnt_sh, sem):
    c = lax.axis_index("c")
    s = lax.axis_index("s")
    wid = s * NC + c

    @pl.loop(0, 8)
    def _fill_ones(i):
        onesv[pl.ds(i * 16, 16)] = jnp.full((16,), 1.0, jnp.float32)

    @pl.loop(0, 50)
    def _fill_zeros(i):
        zb[pl.ds(i * 16, 16)] = jnp.zeros((16,), jnp.float32)

    base = s * RPS

    @pl.loop(0, RPS // 800)
    def _zero(k):
        pltpu.sync_copy(zb, cnt_sh.at[pl.ds(base + k * 800, 800)])

    plsc.subcore_barrier()

    @pl.loop(wid, NCHUNK, step=NW)
    def _scatter(t):
        eb = t * KC
        pltpu.sync_copy(dst_hbm.at[pl.ds(eb, KC)], dstv)
        pltpu.sync_copy(onesv, cnt_sh.at[dstv], add=True)

    plsc.subcore_barrier()

    @pl.loop(0, RPS // 800)
    def _drain(k):
        o = base + k * 800
        pltpu.sync_copy(cnt_sh.at[pl.ds(o, 800)], zb)
        pltpu.sync_copy(zb, out_hbm.at[pl.ds(c * NPAD + o, 800)])


_count_kernel = functools.partial(
    pl.kernel,
    out_type=jax.ShapeDtypeStruct((NC * NPAD,), jnp.float32),
    mesh=_mesh,
    compiler_params=_sc_params,
    scratch_types=[
        pltpu.VMEM((KC,), jnp.int32),
        pltpu.VMEM((KC,), jnp.float32),
        pltpu.VMEM((800,), jnp.float32),
        pltpu.VMEM_SHARED((NPAD,), jnp.float32),
        pltpu.SemaphoreType.DMA,
    ],
)(_cnt_body)


NSLOT_AGG = 2
NSLOT_H1 = 4


def _compute_h(asv, bdv, cv, hv, packed_out):
    @pl.loop(0, K, unroll=8)
    def _rows(i):
        r = i // 4
        cb = (i % 4) * 32
        for j in (0, 16):
            v = jnp.maximum(
                asv[i, pl.ds(j, 16)] + bdv[i, pl.ds(j, 16)]
                + cv[r, pl.ds(cb + j, 16)], 0.0)
            if packed_out:
                hv[r, pl.ds(cb + j, 16)] = v
            else:
                hv[i, pl.ds(j, 16)] = v


def _snapshot_dst(iv, dsc):
    @pl.loop(0, K // 16)
    def _cp(q):
        dsc[pl.ds(q * 16, 16)] = iv[1, pl.ds(q * 16, 16)]


def _c4_slice(c_hbm, t):
    return c_hbm.at[pl.ds((t // 64) * BE4 + (t % 64) * 28, 28)]


def _start_gathers(a_hbm, b_hbm, c_hbm, iv, asv, bdv, cv, gsem, t):
    pltpu.async_copy(a_hbm.at[iv.at[0]], asv, gsem)
    pltpu.async_copy(b_hbm.at[iv.at[1]], bdv, gsem)
    pltpu.async_copy(_c4_slice(c_hbm, t), cv, gsem)


def _wait_gathers(a_hbm, b_hbm, c_hbm, iv, asv, bdv, cv, gsem, t):
    pltpu.make_async_copy(a_hbm.at[iv.at[0]], asv, gsem).wait()
    pltpu.make_async_copy(b_hbm.at[iv.at[1]], bdv, gsem).wait()
    pltpu.make_async_copy(_c4_slice(c_hbm, t), cv, gsem).wait()


def _slot_scratch(nslot, packed_hv):
    hv_shape = (28, 128) if packed_hv else (K, H)
    return [
        pltpu.VMEM((nslot, 2, K), jnp.int32),
        [pltpu.VMEM((K,), jnp.int32) for _ in range(nslot)],
        [pltpu.VMEM((K, D), jnp.float32) for _ in range(nslot)],
        [pltpu.VMEM((K, D), jnp.float32) for _ in range(nslot)],
        [pltpu.VMEM((28, 128), jnp.float32) for _ in range(nslot)],
        [pltpu.VMEM(hv_shape, jnp.float32) for _ in range(nslot)],
        [pltpu.SemaphoreType.DMA for _ in range(nslot)],
        [pltpu.SemaphoreType.DMA for _ in range(nslot)],
    ]


def _agg_body(a_hbm, b_hbm, c_hbm, sd_hbm, out_hbm,
              idxv, dsc, asv, bdv, cv, hv, gsem, ssem, agg_sh):
    c = lax.axis_index("c")
    s = lax.axis_index("s")
    wid = s * NC + c
    nslot = NSLOT_AGG
    niter = CPW // nslot

    @pl.loop(0, K * 2)
    def _fill_zeros(q):
        hv[0][q // 2, pl.ds((q % 2) * 16, 16)] = jnp.zeros((16,), jnp.float32)

    rbase = s * RPA

    @pl.loop(0, 27)
    def _zero(k):
        pltpu.sync_copy(hv[0], agg_sh.at[pl.ds(rbase + k * K, K)])

    pltpu.sync_copy(hv[0].at[pl.ds(0, 104)],
                    agg_sh.at[pl.ds(rbase + 27 * K, 104)])

    plsc.subcore_barrier()

    for i in range(nslot):
        t = wid + NW * i
        pltpu.sync_copy(sd_hbm.at[t], idxv.at[i])
        _start_gathers(a_hbm, b_hbm, c_hbm, idxv.at[i], asv[i], bdv[i],
                       cv[i], gsem[i], t)

    @pl.loop(0, niter)
    def _iter(j):
        for i in range(nslot):
            t = wid + NW * (nslot * j + i)
            _wait_gathers(a_hbm, b_hbm, c_hbm, idxv.at[i], asv[i], bdv[i],
                          cv[i], gsem[i], t)

            @pl.when(j > 0)
            def _wait_prev_scatter():
                pltpu.make_async_copy(hv[i], agg_sh.at[dsc[i]], ssem[i]).wait()

            _compute_h(asv[i], bdv[i], cv[i], hv[i], False)
            _snapshot_dst(idxv.at[i], dsc[i])
            pltpu.async_copy(hv[i], agg_sh.at[dsc[i]], ssem[i], add=True)

            @pl.when(j < niter - 1)
            def _prefetch():
                t2 = wid + NW * (nslot * (j + 1) + i)
                pltpu.sync_copy(sd_hbm.at[t2], idxv.at[i])
                _start_gathers(a_hbm, b_hbm, c_hbm, idxv.at[i], asv[i],
                               bdv[i], cv[i], gsem[i], t2)

    for i in range(nslot):
        pltpu.make_async_copy(hv[i], agg_sh.at[dsc[i]], ssem[i]).wait()

    plsc.subcore_barrier()

    @pl.loop(0, 27)
    def _drain(k):
        o = rbase + k * K
        pltpu.sync_copy(agg_sh.at[pl.ds(o, K)], hv[0])
        pltpu.sync_copy(hv[0], out_hbm.at[c, pl.ds(o, K)])

    o = rbase + 27 * K
    pltpu.sync_copy(agg_sh.at[pl.ds(o, 104)], hv[0].at[pl.ds(0, 104)])
    pltpu.sync_copy(hv[0].at[pl.ds(0, 104)], out_hbm.at[c, pl.ds(o, 104)])


_agg_kernel = functools.partial(
    pl.kernel,
    out_type=jax.ShapeDtypeStruct((NC, NPAD_A, H), jnp.float32),
    mesh=_mesh,
    compiler_params=_sc_params,
    scratch_types=_slot_scratch(NSLOT_AGG, False) + [
        pltpu.VMEM_SHARED((NPAD_A, H), jnp.float32),
    ],
)(_agg_body)


def _h1_body(a_hbm, b_hbm, c_hbm, sd_hbm, out_hbm,
             idxv, dsc, asv, bdv, cv, hv, gsem, ssem):
    c = lax.axis_index("c")
    s = lax.axis_index("s")
    wid = s * NC + c
    nslot = NSLOT_H1
    niter = CPW // nslot

    for i in range(nslot):
        t = wid + NW * i
        pltpu.sync_copy(sd_hbm.at[t], idxv.at[i])
        _start_gathers(a_hbm, b_hbm, c_hbm, idxv.at[i], asv[i], bdv[i],
                       cv[i], gsem[i], t)

    @pl.loop(0, niter)
    def _iter(j):
        for i in range(nslot):
            t = wid + NW * (nslot * j + i)
            _wait_gathers(a_hbm, b_hbm, c_hbm, idxv.at[i], asv[i], bdv[i],
                          cv[i], gsem[i], t)

            @pl.when(j > 0)
            def _wait_prev_store():
                pltpu.make_async_copy(hv[i], _c4_slice(out_hbm, t),
                                      ssem[i]).wait()

            _compute_h(asv[i], bdv[i], cv[i], hv[i], True)
            pltpu.async_copy(hv[i], _c4_slice(out_hbm, t), ssem[i])

            @pl.when(j < niter - 1)
            def _prefetch():
                t2 = wid + NW * (nslot * (j + 1) + i)
                pltpu.sync_copy(sd_hbm.at[t2], idxv.at[i])
                _start_gathers(a_hbm, b_hbm, c_hbm, idxv.at[i], asv[i],
                               bdv[i], cv[i], gsem[i], t2)

    for i in range(nslot):
        pltpu.make_async_copy(hv[i], _c4_slice(out_hbm, 0), ssem[i]).wait()


_h1_kernel = functools.partial(
    pl.kernel,
    out_type=jax.ShapeDtypeStruct((M4, 128), jnp.float32),
    mesh=_mesh,
    compiler_params=_sc_params,
    scratch_types=_slot_scratch(NSLOT_H1, True),
)(_h1_body)


BN = 2000
BEC = 7168


def _embed_body(nf_ref, we_ref, be_ref, w1a_ref, w1b_ref, x_ref, a_ref, b_ref):
    x = jnp.maximum(
        jnp.dot(nf_ref[...], we_ref[...], preferred_element_type=jnp.float32)
        + be_ref[...], 0.0)
    x_ref[...] = x
    a_ref[...] = jnp.dot(x, w1a_ref[...], preferred_element_type=jnp.float32)
    b_ref[...] = jnp.dot(x, w1b_ref[...], preferred_element_type=jnp.float32)


def _embed_call(nf, we, be2, w1a, w1b):
    n32 = jax.ShapeDtypeStruct((N, D), jnp.float32)
    return pl.pallas_call(
        _embed_body,
        grid=(N // BN,),
        in_specs=[
            pl.BlockSpec((BN, 7), lambda i: (i, 0)),
            pl.BlockSpec((7, D), lambda i: (0, 0)),
            pl.BlockSpec((1, D), lambda i: (0, 0)),
            pl.BlockSpec((D, D), lambda i: (0, 0)),
            pl.BlockSpec((D, D), lambda i: (0, 0)),
        ],
        out_specs=[pl.BlockSpec((BN, D), lambda i: (i, 0))] * 3,
        out_shape=[n32, n32, n32],
    )(nf, we, be2, w1a, w1b)


def _edgec_body(*refs):
    nout = (len(refs) - 6)
    ea_refs, w_ref, b_ref, crefs = refs[:4], refs[4], refs[5], refs[6:]
    eas = [r[...] for r in ea_refs]
    i = pl.program_id(0)
    iota = lax.broadcasted_iota(jnp.int32, (BE4, 1), 0)
    valids = [(4 * i + j) * BE4 + iota < E for j in range(4)]
    for l, cref in enumerate(crefs):
        parts = []
        for j in range(4):
            v = jnp.dot(eas[j], w_ref[l],
                        preferred_element_type=jnp.float32) + b_ref[l]
            parts.append(jnp.where(valids[j], v, -1e30))
        cref[...] = jnp.concatenate(parts, axis=1)


def _edgec_call(ea, wstack, bstack):
    nout = wstack.shape[0]
    e32 = jax.ShapeDtypeStruct((M4, 128), jnp.float32)
    last = (E - 1) // BE4

    def easpec(j):
        return pl.BlockSpec(
            (BE4, 16), lambda i, j=j: (jnp.minimum(4 * i + j, last), 0))

    return pl.pallas_call(
        _edgec_body,
        grid=(GE,),
        in_specs=[
            easpec(0), easpec(1), easpec(2), easpec(3),
            pl.BlockSpec((nout, 16, H), lambda i: (0, 0, 0)),
            pl.BlockSpec((nout, H), lambda i: (0, 0)),
        ],
        out_specs=[pl.BlockSpec((BE4, 128), lambda i: (i, 0))] * nout,
        out_shape=[e32] * nout,
    )(ea, ea, ea, ea, wstack, bstack)


def _update_body(x_ref, agg2_ref, cnt2_ref, wm2_ref, bm2_ref, wua_ref, wub_ref,
                 bu_ref, g_ref, be_ref, wna_ref, wnb_ref,
                 xo_ref, ao_ref, bo_ref):
    x = x_ref[...]
    aggs = agg2_ref[0] + agg2_ref[1]
    cnt = cnt2_ref[0] + cnt2_ref[1]
    msum = jnp.dot(aggs, wm2_ref[...], preferred_element_type=jnp.float32) \
        + cnt * bm2_ref[...]
    agg = msum / (cnt + 1e-6)
    upd = jnp.maximum(
        jnp.dot(x, wua_ref[...], preferred_element_type=jnp.float32)
        + jnp.dot(agg, wub_ref[...], preferred_element_type=jnp.float32)
        + bu_ref[...], 0.0)
    xn = x + upd
    mu = jnp.mean(xn, axis=-1, keepdims=True)
    var = jnp.mean((xn - mu) ** 2, axis=-1, keepdims=True)
    xn = (xn - mu) / jnp.sqrt(var + 1e-5) * g_ref[...] + be_ref[...]
    xo_ref[...] = xn
    ao_ref[...] = jnp.dot(xn, wna_ref[...], preferred_element_type=jnp.float32)
    bo_ref[...] = jnp.dot(xn, wnb_ref[...], preferred_element_type=jnp.float32)


def _update_call(x, agg2, cnt2, wm2, bm2_2, wua, wub, bu2, g2, be2, wna, wnb):
    n32 = jax.ShapeDtypeStruct((N, D), jnp.float32)
    wspec = pl.BlockSpec((D, D), lambda i: (0, 0))
    vspec = pl.BlockSpec((1, D), lambda i: (0, 0))
    return pl.pallas_call(
        _update_body,
        grid=(N // BN,),
        in_specs=[
            pl.BlockSpec((BN, D), lambda i: (i, 0)),
            pl.BlockSpec((NC, BN, H), lambda i: (0, i, 0)),
            pl.BlockSpec((NC, BN, 1), lambda i: (0, i, 0)),
            wspec, vspec, wspec, wspec, vspec, vspec, vspec, wspec, wspec,
        ],
        out_specs=[pl.BlockSpec((BN, D), lambda i: (i, 0))] * 3,
        out_shape=[n32, n32, n32],
    )(x, agg2, cnt2, wm2, bm2_2, wua, wub, bu2, g2, be2, wna, wnb)


def _pred_body(h4_ref, wp2_ref, bp2_ref, wp3_ref, bp3_ref, out_ref):
    h2 = jnp.maximum(
        jnp.dot(h4_ref[...], wp2_ref[...], preferred_element_type=jnp.float32)
        + bp2_ref[...], 0.0)
    z = jnp.dot(h2, wp3_ref[...], preferred_element_type=jnp.float32) \
        + bp3_ref[...]
    out_ref[...] = jnp.tanh(z)


def _pred_call(h4, wp2_4, bp2_4, wp3_4, bp3_4):
    return pl.pallas_call(
        _pred_body,
        grid=(GE,),
        in_specs=[
            pl.BlockSpec((BE4, 128), lambda i: (i, 0)),
            pl.BlockSpec((128, 64), lambda i: (0, 0)),
            pl.BlockSpec((1, 64), lambda i: (0, 0)),
            pl.BlockSpec((64, 4), lambda i: (0, 0)),
            pl.BlockSpec((1, 4), lambda i: (0, 0)),
        ],
        out_specs=pl.BlockSpec((BE4, 4), lambda i: (i, 0)),
        out_shape=jax.ShapeDtypeStruct((M4, 4), jnp.float32),
    )(h4, wp2_4, bp2_4, wp3_4, bp3_4)


def kernel(node_features, edge_index, edge_attr, W_embed, b_embed,
           Wm1_0, bm1_0, Wm2_0, bm2_0, Wu_0, bu_0, g_0, be_0,
           Wm1_1, bm1_1, Wm2_1, bm2_1, Wu_1, bu_1, g_1, be_1,
           Wm1_2, bm1_2, Wm2_2, bm2_2, Wu_2, bu_2, g_2, be_2,
           Wp1, bp1, Wp2, bp2, Wp3, bp3):
    src = edge_index[0]
    dst = edge_index[1]

    pad = E_PAD - E
    src_p = jnp.concatenate([src, jnp.zeros((pad,), jnp.int32)])
    dst_p = jnp.concatenate([dst, jnp.zeros((pad,), jnp.int32)])
    def _chunk_perm(v):
        return v.reshape(GE, 4, 64, 28).transpose(0, 2, 3, 1).reshape(
            NCHUNK_PAD, K)

    sd = jnp.stack([_chunk_perm(src_p), _chunk_perm(dst_p)], axis=1)

    wm1 = [Wm1_0, Wm1_1, Wm1_2]
    bm1 = [bm1_0, bm1_1, bm1_2]
    wm2 = [Wm2_0, Wm2_1, Wm2_2]
    bm2 = [bm2_0, bm2_1, bm2_2]
    wu = [Wu_0, Wu_1, Wu_2]
    bu = [bu_0, bu_1, bu_2]
    g = [g_0, g_1, g_2]
    be = [be_0, be_1, be_2]

    cnt2 = _count_kernel(dst).reshape(NC, NPAD, 1)

    (c0,) = _edgec_call(edge_attr, wm1[0][2 * D:][None], bm1[0][None])
    wstack = jnp.stack([wm1[1][2 * D:], wm1[2][2 * D:], Wp1[2 * D:]])
    bstack = jnp.stack([bm1[1], bm1[2], bp1])
    c123 = _edgec_call(edge_attr, wstack, bstack)
    c_all = [c0] + list(c123)

    x, a, b = _embed_call(node_features, W_embed, b_embed[None, :],
                          wm1[0][:D], wm1[0][D:2 * D])

    for l in range(3):
        agg2 = _agg_kernel(a, b, c_all[l], sd)
        if l < 2:
            wna, wnb = wm1[l + 1][:D], wm1[l + 1][D:2 * D]
        else:
            wna, wnb = Wp1[:D], Wp1[D:2 * D]
        x, a, b = _update_call(
            x, agg2, cnt2, wm2[l], bm2[l][None, :], wu[l][:D], wu[l][D:],
            bu[l][None, :], g[l][None, :], be[l][None, :], wna, wnb)

    zb2 = jnp.zeros((D, H // 2), jnp.float32)
    wp2_4 = jnp.concatenate([
        jnp.concatenate([Wp2 if r == c else zb2 for c in range(4)], axis=1)
        for r in range(4)], axis=0)
    zb3 = jnp.zeros((H // 2, 1), jnp.float32)
    wp3_4 = jnp.concatenate([
        jnp.concatenate([Wp3 if r == c else zb3 for c in range(4)], axis=1)
        for r in range(4)], axis=0)
    bp2_4 = jnp.tile(bp2, 4)[None, :]
    bp3_4 = jnp.tile(bp3, 4)[None, :]

    h4 = _h1_kernel(a, b, c_all[3], sd)
    p4 = _pred_call(h4, wp2_4, bp2_4, wp3_4, bp3_4)
    p = p4.reshape(GE, BE4, 4).transpose(0, 2, 1).reshape(E_PAD)
    return p[:E]


# speedup vs baseline: 5.9778x; 1.0489x over previous
import functools

import jax
import jax.numpy as jnp
from jax import lax
from jax.experimental import pallas as pl
from jax.experimental.pallas import tpu as pltpu
from jax.experimental.pallas import tpu_sc as plsc

N = 50000
E = 800000
D = 32
H = 32

NC = 2
NS = 16

KC = 128
NCHUNK = E // KC
NW = NC * NS
K = 112
CPW = 224
NCHUNK_PAD = NW * CPW
E_PAD = NCHUNK_PAD * K
NPAD = 51200
RPS = NPAD // NS
NPAD_A = 50048
RPA = NPAD_A // NS
M4 = E_PAD // 4
BE4 = 1792
GE = E_PAD // (4 * BE4)

_mesh = plsc.VectorSubcoreMesh(core_axis_name="c", subcore_axis_name="s")
_sc_params = pltpu.CompilerParams(use_tc_tiling_on_sc=False)


def _cnt_body(dst_hbm, out_hbm, dstv, onesv, zb, cnt_sh, sem):
    c = lax.axis_index("c")
    s = lax.axis_index("s")
    wid = s * NC + c

    @pl.loop(0, 8)
    def _fill_ones(i):
        onesv[pl.ds(i * 16, 16)] = jnp.full((16,), 1.0, jnp.float32)

    @pl.loop(0, 50)
    def _fill_zeros(i):
        zb[pl.ds(i * 16, 16)] = jnp.zeros((16,), jnp.float32)

    base = s * RPS

    @pl.loop(0, RPS // 800)
    def _zero(k):
        pltpu.sync_copy(zb, cnt_sh.at[pl.ds(base + k * 800, 800)])

    plsc.subcore_barrier()

    @pl.loop(wid, NCHUNK, step=NW)
    def _scatter(t):
        eb = t * KC
        pltpu.sync_copy(dst_hbm.at[pl.ds(eb, KC)], dstv)
        pltpu.sync_copy(onesv, cnt_sh.at[dstv], add=True)

    plsc.subcore_barrier()

    @pl.loop(0, RPS // 800)
    def _drain(k):
        o = base + k * 800
        pltpu.sync_copy(cnt_sh.at[pl.ds(o, 800)], zb)
        pltpu.sync_copy(zb, out_hbm.at[pl.ds(c * NPAD + o, 800)])


_count_kernel = functools.partial(
    pl.kernel,
    out_type=jax.ShapeDtypeStruct((NC * NPAD,), jnp.float32),
    mesh=_mesh,
    compiler_params=_sc_params,
    scratch_types=[
        pltpu.VMEM((KC,), jnp.int32),
        pltpu.VMEM((KC,), jnp.float32),
        pltpu.VMEM((800,), jnp.float32),
        pltpu.VMEM_SHARED((NPAD,), jnp.float32),
        pltpu.SemaphoreType.DMA,
    ],
)(_cnt_body)


NSLOT_AGG = 2
NSLOT_H1 = 4


def _unpack_cols(w):
    lo = jax.lax.bitcast_convert_type(w << 16, jnp.float32)
    hi = jax.lax.bitcast_convert_type(w & jnp.int32(-65536), jnp.float32)
    return lo, hi


def _compute_h(asv, bdv, cv, hv, packed_out):
    @pl.loop(0, K, unroll=8)
    def _rows(i):
        r = i // 4
        cb = (i % 4) * 32
        a01 = _unpack_cols(asv[i, pl.ds(0, 16)])
        b01 = _unpack_cols(bdv[i, pl.ds(0, 16)])
        for half, j in ((0, 0), (1, 16)):
            v = jnp.maximum(
                a01[half] + b01[half] + cv[r, pl.ds(cb + j, 16)], 0.0)
            if packed_out:
                hv[r, pl.ds(cb + j, 16)] = v
            else:
                hv[i, pl.ds(j, 16)] = v


def _snapshot_dst(iv, dsc):
    @pl.loop(0, K // 16)
    def _cp(q):
        dsc[pl.ds(q * 16, 16)] = iv[1, pl.ds(q * 16, 16)]


def _c4_slice(c_hbm, t):
    return c_hbm.at[pl.ds((t // 64) * BE4 + (t % 64) * 28, 28)]


def _start_gathers(a_hbm, b_hbm, c_hbm, iv, asv, bdv, cv, gsem, t):
    pltpu.async_copy(a_hbm.at[iv.at[0]], asv, gsem)
    pltpu.async_copy(b_hbm.at[iv.at[1]], bdv, gsem)
    pltpu.async_copy(_c4_slice(c_hbm, t), cv, gsem)


def _wait_gathers(a_hbm, b_hbm, c_hbm, iv, asv, bdv, cv, gsem, t):
    pltpu.make_async_copy(a_hbm.at[iv.at[0]], asv, gsem).wait()
    pltpu.make_async_copy(b_hbm.at[iv.at[1]], bdv, gsem).wait()
    pltpu.make_async_copy(_c4_slice(c_hbm, t), cv, gsem).wait()


def _slot_scratch(nslot, packed_hv):
    hv_shape = (28, 128) if packed_hv else (K, H)
    return [
        pltpu.VMEM((nslot, 2, K), jnp.int32),
        [pltpu.VMEM((K,), jnp.int32) for _ in range(nslot)],
        [pltpu.VMEM((K, 16), jnp.int32) for _ in range(nslot)],
        [pltpu.VMEM((K, 16), jnp.int32) for _ in range(nslot)],
        [pltpu.VMEM((28, 128), jnp.float32) for _ in range(nslot)],
        [pltpu.VMEM(hv_shape, jnp.float32) for _ in range(nslot)],
        [pltpu.SemaphoreType.DMA for _ in range(nslot)],
        [pltpu.SemaphoreType.DMA for _ in range(nslot)],
    ]


def _agg_body(a_hbm, b_hbm, c_hbm, sd_hbm, out_hbm,
              idxv, dsc, asv, bdv, cv, hv, gsem, ssem, agg_sh):
    c = lax.axis_index("c")
    s = lax.axis_index("s")
    wid = s * NC + c
    nslot = NSLOT_AGG
    niter = CPW // nslot

    @pl.loop(0, K * 2)
    def _fill_zeros(q):
        hv[0][q // 2, pl.ds((q % 2) * 16, 16)] = jnp.zeros((16,), jnp.float32)

    rbase = s * RPA

    @pl.loop(0, 27)
    def _zero(k):
        pltpu.sync_copy(hv[0], agg_sh.at[pl.ds(rbase + k * K, K)])

    pltpu.sync_copy(hv[0].at[pl.ds(0, 104)],
                    agg_sh.at[pl.ds(rbase + 27 * K, 104)])

    plsc.subcore_barrier()

    for i in range(nslot):
        t = wid + NW * i
        pltpu.sync_copy(sd_hbm.at[t], idxv.at[i])
        _start_gathers(a_hbm, b_hbm, c_hbm, idxv.at[i], asv[i], bdv[i],
                       cv[i], gsem[i], t)

    @pl.loop(0, niter)
    def _iter(j):
        for i in range(nslot):
            t = wid + NW * (nslot * j + i)
            _wait_gathers(a_hbm, b_hbm, c_hbm, idxv.at[i], asv[i], bdv[i],
                          cv[i], gsem[i], t)

            @pl.when(j > 0)
            def _wait_prev_scatter():
                pltpu.make_async_copy(hv[i], agg_sh.at[dsc[i]], ssem[i]).wait()

            _compute_h(asv[i], bdv[i], cv[i], hv[i], False)
            _snapshot_dst(idxv.at[i], dsc[i])
            pltpu.async_copy(hv[i], agg_sh.at[dsc[i]], ssem[i], add=True)

            @pl.when(j < niter - 1)
            def _prefetch():
                t2 = wid + NW * (nslot * (j + 1) + i)
                pltpu.sync_copy(sd_hbm.at[t2], idxv.at[i])
                _start_gathers(a_hbm, b_hbm, c_hbm, idxv.at[i], asv[i],
                               bdv[i], cv[i], gsem[i], t2)

    for i in range(nslot):
        pltpu.make_async_copy(hv[i], agg_sh.at[dsc[i]], ssem[i]).wait()

    plsc.subcore_barrier()

    @pl.loop(0, 27)
    def _drain(k):
        o = rbase + k * K
        pltpu.sync_copy(agg_sh.at[pl.ds(o, K)], hv[0])
        pltpu.sync_copy(hv[0], out_hbm.at[c, pl.ds(o, K)])

    o = rbase + 27 * K
    pltpu.sync_copy(agg_sh.at[pl.ds(o, 104)], hv[0].at[pl.ds(0, 104)])
    pltpu.sync_copy(hv[0].at[pl.ds(0, 104)], out_hbm.at[c, pl.ds(o, 104)])


_agg_kernel = functools.partial(
    pl.kernel,
    out_type=jax.ShapeDtypeStruct((NC, NPAD_A, H), jnp.float32),
    mesh=_mesh,
    compiler_params=_sc_params,
    scratch_types=_slot_scratch(NSLOT_AGG, False) + [
        pltpu.VMEM_SHARED((NPAD_A, H), jnp.float32),
    ],
)(_agg_body)


def _h1_body(a_hbm, b_hbm, c_hbm, sd_hbm, out_hbm,
             idxv, dsc, asv, bdv, cv, hv, gsem, ssem):
    c = lax.axis_index("c")
    s = lax.axis_index("s")
    wid = s * NC + c
    nslot = NSLOT_H1
    niter = CPW // nslot

    for i in range(nslot):
        t = wid + NW * i
        pltpu.sync_copy(sd_hbm.at[t], idxv.at[i])
        _start_gathers(a_hbm, b_hbm, c_hbm, idxv.at[i], asv[i], bdv[i],
                       cv[i], gsem[i], t)

    @pl.loop(0, niter)
    def _iter(j):
        for i in range(nslot):
            t = wid + NW * (nslot * j + i)
            _wait_gathers(a_hbm, b_hbm, c_hbm, idxv.at[i], asv[i], bdv[i],
                          cv[i], gsem[i], t)

            @pl.when(j > 0)
            def _wait_prev_store():
                pltpu.make_async_copy(hv[i], _c4_slice(out_hbm, t),
                                      ssem[i]).wait()

            _compute_h(asv[i], bdv[i], cv[i], hv[i], True)
            pltpu.async_copy(hv[i], _c4_slice(out_hbm, t), ssem[i])

            @pl.when(j < niter - 1)
            def _prefetch():
                t2 = wid + NW * (nslot * (j + 1) + i)
                pltpu.sync_copy(sd_hbm.at[t2], idxv.at[i])
                _start_gathers(a_hbm, b_hbm, c_hbm, idxv.at[i], asv[i],
                               bdv[i], cv[i], gsem[i], t2)

    for i in range(nslot):
        pltpu.make_async_copy(hv[i], _c4_slice(out_hbm, 0), ssem[i]).wait()


_h1_kernel = functools.partial(
    pl.kernel,
    out_type=jax.ShapeDtypeStruct((M4, 128), jnp.float32),
    mesh=_mesh,
    compiler_params=_sc_params,
    scratch_types=_slot_scratch(NSLOT_H1, True),
)(_h1_body)


BN = 2000


def _pack_bf16_pair(af):
    bits = jax.lax.bitcast_convert_type(af, jnp.int32)
    rtne = bits + 0x7FFF + ((bits >> 16) & 1)
    lo = (rtne[:, :16] >> 16) & 0xFFFF
    hi = rtne[:, 16:] & jnp.int32(-65536)
    return hi | lo
BEC = 7168


def _embed_body(nf_ref, we_ref, be_ref, w1a_ref, w1b_ref, x_ref, a_ref, b_ref):
    x = jnp.maximum(
        jnp.dot(nf_ref[...], we_ref[...], preferred_element_type=jnp.float32)
        + be_ref[...], 0.0)
    x_ref[...] = x
    a_ref[...] = _pack_bf16_pair(
        jnp.dot(x, w1a_ref[...], preferred_element_type=jnp.float32))
    b_ref[...] = _pack_bf16_pair(
        jnp.dot(x, w1b_ref[...], preferred_element_type=jnp.float32))


def _embed_call(nf, we, be2, w1a, w1b):
    n32 = jax.ShapeDtypeStruct((N, D), jnp.float32)
    nbf = jax.ShapeDtypeStruct((N, 16), jnp.int32)
    return pl.pallas_call(
        _embed_body,
        grid=(N // BN,),
        in_specs=[
            pl.BlockSpec((BN, 7), lambda i: (i, 0)),
            pl.BlockSpec((7, D), lambda i: (0, 0)),
            pl.BlockSpec((1, D), lambda i: (0, 0)),
            pl.BlockSpec((D, D), lambda i: (0, 0)),
            pl.BlockSpec((D, D), lambda i: (0, 0)),
        ],
        out_specs=[pl.BlockSpec((BN, D), lambda i: (i, 0)),
                   pl.BlockSpec((BN, 16), lambda i: (i, 0)),
                   pl.BlockSpec((BN, 16), lambda i: (i, 0))],
        out_shape=[n32, nbf, nbf],
    )(nf, we, be2, w1a, w1b)


def _edgec_body(*refs):
    nout = (len(refs) - 6)
    ea_refs, w_ref, b_ref, crefs = refs[:4], refs[4], refs[5], refs[6:]
    eas = [r[...] for r in ea_refs]
    i = pl.program_id(0)
    iota = lax.broadcasted_iota(jnp.int32, (BE4, 1), 0)
    valids = [(4 * i + j) * BE4 + iota < E for j in range(4)]
    for l, cref in enumerate(crefs):
        parts = []
        for j in range(4):
            v = jnp.dot(eas[j], w_ref[l],
                        preferred_element_type=jnp.float32) + b_ref[l]
            parts.append(jnp.where(valids[j], v, -1e30))
        cref[...] = jnp.concatenate(parts, axis=1)


def _edgec_call(ea, wstack, bstack):
    nout = wstack.shape[0]
    e32 = jax.ShapeDtypeStruct((M4, 128), jnp.float32)
    last = (E - 1) // BE4

    def easpec(j):
        return pl.BlockSpec(
            (BE4, 16), lambda i, j=j: (jnp.minimum(4 * i + j, last), 0))

    return pl.pallas_call(
        _edgec_body,
        grid=(GE,),
        in_specs=[
            easpec(0), easpec(1), easpec(2), easpec(3),
            pl.BlockSpec((nout, 16, H), lambda i: (0, 0, 0)),
            pl.BlockSpec((nout, H), lambda i: (0, 0)),
        ],
        out_specs=[pl.BlockSpec((BE4, 128), lambda i: (i, 0))] * nout,
        out_shape=[e32] * nout,
    )(ea, ea, ea, ea, wstack, bstack)


def _update_body(x_ref, agg2_ref, cnt2_ref, wm2_ref, bm2_ref, wua_ref, wub_ref,
                 bu_ref, g_ref, be_ref, wna_ref, wnb_ref,
                 xo_ref, ao_ref, bo_ref):
    x = x_ref[...]
    aggs = agg2_ref[0] + agg2_ref[1]
    cnt = cnt2_ref[0] + cnt2_ref[1]
    msum = jnp.dot(aggs, wm2_ref[...], preferred_element_type=jnp.float32) \
        + cnt * bm2_ref[...]
    agg = msum / (cnt + 1e-6)
    upd = jnp.maximum(
        jnp.dot(x, wua_ref[...], preferred_element_type=jnp.float32)
        + jnp.dot(agg, wub_ref[...], preferred_element_type=jnp.float32)
        + bu_ref[...], 0.0)
    xn = x + upd
    mu = jnp.mean(xn, axis=-1, keepdims=True)
    var = jnp.mean((xn - mu) ** 2, axis=-1, keepdims=True)
    xn = (xn - mu) / jnp.sqrt(var + 1e-5) * g_ref[...] + be_ref[...]
    xo_ref[...] = xn
    ao_ref[...] = _pack_bf16_pair(
        jnp.dot(xn, wna_ref[...], preferred_element_type=jnp.float32))
    bo_ref[...] = _pack_bf16_pair(
        jnp.dot(xn, wnb_ref[...], preferred_element_type=jnp.float32))


def _update_call(x, agg2, cnt2, wm2, bm2_2, wua, wub, bu2, g2, be2, wna, wnb):
    n32 = jax.ShapeDtypeStruct((N, D), jnp.float32)
    nbf = jax.ShapeDtypeStruct((N, 16), jnp.int32)
    wspec = pl.BlockSpec((D, D), lambda i: (0, 0))
    vspec = pl.BlockSpec((1, D), lambda i: (0, 0))
    return pl.pallas_call(
        _update_body,
        grid=(N // BN,),
        in_specs=[
            pl.BlockSpec((BN, D), lambda i: (i, 0)),
            pl.BlockSpec((NC, BN, H), lambda i: (0, i, 0)),
            pl.BlockSpec((NC, BN, 1), lambda i: (0, i, 0)),
            wspec, vspec, wspec, wspec, vspec, vspec, vspec, wspec, wspec,
        ],
        out_specs=[pl.BlockSpec((BN, D), lambda i: (i, 0)),
                   pl.BlockSpec((BN, 16), lambda i: (i, 0)),
                   pl.BlockSpec((BN, 16), lambda i: (i, 0))],
        out_shape=[n32, nbf, nbf],
    )(x, agg2, cnt2, wm2, bm2_2, wua, wub, bu2, g2, be2, wna, wnb)


def _pred_body(h4_ref, wp2_ref, bp2_ref, wp3_ref, bp3_ref, out_ref):
    h2 = jnp.maximum(
        jnp.dot(h4_ref[...], wp2_ref[...], preferred_element_type=jnp.float32)
        + bp2_ref[...], 0.0)
    z = jnp.dot(h2, wp3_ref[...], preferred_element_type=jnp.float32) \
        + bp3_ref[...]
    out_ref[...] = jnp.tanh(z)


def _pred_call(h4, wp2_4, bp2_4, wp3_4, bp3_4):
    return pl.pallas_call(
        _pred_body,
        grid=(GE,),
        in_specs=[
            pl.BlockSpec((BE4, 128), lambda i: (i, 0)),
            pl.BlockSpec((128, 64), lambda i: (0, 0)),
            pl.BlockSpec((1, 64), lambda i: (0, 0)),
            pl.BlockSpec((64, 4), lambda i: (0, 0)),
            pl.BlockSpec((1, 4), lambda i: (0, 0)),
        ],
        out_specs=pl.BlockSpec((BE4, 4), lambda i: (i, 0)),
        out_shape=jax.ShapeDtypeStruct((M4, 4), jnp.float32),
    )(h4, wp2_4, bp2_4, wp3_4, bp3_4)


def kernel(node_features, edge_index, edge_attr, W_embed, b_embed,
           Wm1_0, bm1_0, Wm2_0, bm2_0, Wu_0, bu_0, g_0, be_0,
           Wm1_1, bm1_1, Wm2_1, bm2_1, Wu_1, bu_1, g_1, be_1,
           Wm1_2, bm1_2, Wm2_2, bm2_2, Wu_2, bu_2, g_2, be_2,
           Wp1, bp1, Wp2, bp2, Wp3, bp3):
    src = edge_index[0]
    dst = edge_index[1]

    pad = E_PAD - E
    src_p = jnp.concatenate([src, jnp.zeros((pad,), jnp.int32)])
    dst_p = jnp.concatenate([dst, jnp.zeros((pad,), jnp.int32)])
    def _chunk_perm(v):
        return v.reshape(GE, 4, 64, 28).transpose(0, 2, 3, 1).reshape(
            NCHUNK_PAD, K)

    sd = jnp.stack([_chunk_perm(src_p), _chunk_perm(dst_p)], axis=1)

    wm1 = [Wm1_0, Wm1_1, Wm1_2]
    bm1 = [bm1_0, bm1_1, bm1_2]
    wm2 = [Wm2_0, Wm2_1, Wm2_2]
    bm2 = [bm2_0, bm2_1, bm2_2]
    wu = [Wu_0, Wu_1, Wu_2]
    bu = [bu_0, bu_1, bu_2]
    g = [g_0, g_1, g_2]
    be = [be_0, be_1, be_2]

    cnt2 = _count_kernel(dst).reshape(NC, NPAD, 1)

    (c0,) = _edgec_call(edge_attr, wm1[0][2 * D:][None], bm1[0][None])
    wstack = jnp.stack([wm1[1][2 * D:], wm1[2][2 * D:], Wp1[2 * D:]])
    bstack = jnp.stack([bm1[1], bm1[2], bp1])
    c123 = _edgec_call(edge_attr, wstack, bstack)
    c_all = [c0] + list(c123)

    x, a, b = _embed_call(node_features, W_embed, b_embed[None, :],
                          wm1[0][:D], wm1[0][D:2 * D])

    for l in range(3):
        agg2 = _agg_kernel(a, b, c_all[l], sd)
        if l < 2:
            wna, wnb = wm1[l + 1][:D], wm1[l + 1][D:2 * D]
        else:
            wna, wnb = Wp1[:D], Wp1[D:2 * D]
        x, a, b = _update_call(
            x, agg2, cnt2, wm2[l], bm2[l][None, :], wu[l][:D], wu[l][D:],
            bu[l][None, :], g[l][None, :], be[l][None, :], wna, wnb)

    zb2 = jnp.zeros((D, H // 2), jnp.float32)
    wp2_4 = jnp.concatenate([
        jnp.concatenate([Wp2 if r == c else zb2 for c in range(4)], axis=1)
        for r in range(4)], axis=0)
    zb3 = jnp.zeros((H // 2, 1), jnp.float32)
    wp3_4 = jnp.concatenate([
        jnp.concatenate([Wp3 if r == c else zb3 for c in range(4)], axis=1)
        for r in range(4)], axis=0)
    bp2_4 = jnp.tile(bp2, 4)[None, :]
    bp3_4 = jnp.tile(bp3, 4)[None, :]

    h4 = _h1_kernel(a, b, c_all[3], sd)
    p4 = _pred_call(h4, wp2_4, bp2_4, wp3_4, bp3_4)
    p = p4.reshape(GE, BE4, 4).transpose(0, 2, 1).reshape(E_PAD)
    return p[:E]


# speedup vs baseline: 6.1657x; 1.0314x over previous
import functools

import jax
import jax.numpy as jnp
from jax import lax
from jax.experimental import pallas as pl
from jax.experimental.pallas import tpu as pltpu
from jax.experimental.pallas import tpu_sc as plsc

N = 50000
E = 800000
D = 32
H = 32

NC = 2
NS = 16

KC = 128
NCHUNK = E // KC
NW = NC * NS
K = 128
CPW = 196
NCHUNK_PAD = NW * CPW
E_PAD = NCHUNK_PAD * K
NPAD = 51200
RPS = NPAD // NS
NPAD_A = 50048
RPA = NPAD_A // NS
M4 = E_PAD // 4
BE4 = 1792
GE = E_PAD // (4 * BE4)

_mesh = plsc.VectorSubcoreMesh(core_axis_name="c", subcore_axis_name="s")
_sc_params = pltpu.CompilerParams(use_tc_tiling_on_sc=False)


def _cnt_body(dst_hbm, out_hbm, dstv, onesv, zb, cnt_sh, sem):
    c = lax.axis_index("c")
    s = lax.axis_index("s")
    wid = s * NC + c

    @pl.loop(0, 8)
    def _fill_ones(i):
        onesv[pl.ds(i * 16, 16)] = jnp.full((16,), 1.0, jnp.float32)

    @pl.loop(0, 50)
    def _fill_zeros(i):
        zb[pl.ds(i * 16, 16)] = jnp.zeros((16,), jnp.float32)

    base = s * RPS

    @pl.loop(0, RPS // 800)
    def _zero(k):
        pltpu.sync_copy(zb, cnt_sh.at[pl.ds(base + k * 800, 800)])

    plsc.subcore_barrier()

    @pl.loop(wid, NCHUNK, step=NW)
    def _scatter(t):
        eb = t * KC
        pltpu.sync_copy(dst_hbm.at[pl.ds(eb, KC)], dstv)
        pltpu.sync_copy(onesv, cnt_sh.at[dstv], add=True)

    plsc.subcore_barrier()

    @pl.loop(0, RPS // 800)
    def _drain(k):
        o = base + k * 800
        pltpu.sync_copy(cnt_sh.at[pl.ds(o, 800)], zb)
        pltpu.sync_copy(zb, out_hbm.at[pl.ds(c * NPAD + o, 800)])


_count_kernel = functools.partial(
    pl.kernel,
    out_type=jax.ShapeDtypeStruct((NC * NPAD,), jnp.float32),
    mesh=_mesh,
    compiler_params=_sc_params,
    scratch_types=[
        pltpu.VMEM((KC,), jnp.int32),
        pltpu.VMEM((KC,), jnp.float32),
        pltpu.VMEM((800,), jnp.float32),
        pltpu.VMEM_SHARED((NPAD,), jnp.float32),
        pltpu.SemaphoreType.DMA,
    ],
)(_cnt_body)


NSLOT_AGG = 2
NSLOT_H1 = 4


def _unpack_cols(w):
    lo = jax.lax.bitcast_convert_type(w << 16, jnp.float32)
    hi = jax.lax.bitcast_convert_type(w & jnp.int32(-65536), jnp.float32)
    return lo, hi


def _compute_h(asv, bdv, cv, hv, packed_out):
    @pl.loop(0, K, unroll=8)
    def _rows(i):
        r = i // 4
        cb = (i % 4) * 32
        a01 = _unpack_cols(asv[i, pl.ds(0, 16)])
        b01 = _unpack_cols(bdv[i, pl.ds(0, 16)])
        for half, j in ((0, 0), (1, 16)):
            v = jnp.maximum(
                a01[half] + b01[half] + cv[r, pl.ds(cb + j, 16)], 0.0)
            if packed_out:
                hv[r, pl.ds(cb + j, 16)] = v
            else:
                hv[i, pl.ds(j, 16)] = v


def _snapshot_dst(iv, dsc):
    @pl.loop(0, K // 16)
    def _cp(q):
        dsc[pl.ds(q * 16, 16)] = iv[1, pl.ds(q * 16, 16)]


def _c4_slice(c_hbm, t):
    return c_hbm.at[pl.ds((t // 56) * BE4 + (t % 56) * 32, 32)]


def _start_gathers(a_hbm, b_hbm, c_hbm, iv, asv, bdv, cv, gsem, t):
    pltpu.async_copy(a_hbm.at[iv.at[0]], asv, gsem)
    pltpu.async_copy(b_hbm.at[iv.at[1]], bdv, gsem)
    pltpu.async_copy(_c4_slice(c_hbm, t), cv, gsem)


def _wait_gathers(a_hbm, b_hbm, c_hbm, iv, asv, bdv, cv, gsem, t):
    pltpu.make_async_copy(a_hbm.at[iv.at[0]], asv, gsem).wait()
    pltpu.make_async_copy(b_hbm.at[iv.at[1]], bdv, gsem).wait()
    pltpu.make_async_copy(_c4_slice(c_hbm, t), cv, gsem).wait()


def _slot_scratch(nslot, packed_hv):
    hv_shape = (32, 128) if packed_hv else (K, H)
    return [
        pltpu.VMEM((nslot, 2, K), jnp.int32),
        [pltpu.VMEM((K,), jnp.int32) for _ in range(nslot)],
        [pltpu.VMEM((K, 16), jnp.int32) for _ in range(nslot)],
        [pltpu.VMEM((K, 16), jnp.int32) for _ in range(nslot)],
        [pltpu.VMEM((32, 128), jnp.float32) for _ in range(nslot)],
        [pltpu.VMEM(hv_shape, jnp.float32) for _ in range(nslot)],
        [pltpu.SemaphoreType.DMA for _ in range(nslot)],
        [pltpu.SemaphoreType.DMA for _ in range(nslot)],
    ]


def _agg_body(a_hbm, b_hbm, c_hbm, sd_hbm, out_hbm,
              idxv, dsc, asv, bdv, cv, hv, gsem, ssem, agg_sh):
    c = lax.axis_index("c")
    s = lax.axis_index("s")
    wid = s * NC + c
    nslot = NSLOT_AGG
    niter = CPW // nslot

    @pl.loop(0, K * 2)
    def _fill_zeros(q):
        hv[0][q // 2, pl.ds((q % 2) * 16, 16)] = jnp.zeros((16,), jnp.float32)

    rbase = s * RPA

    @pl.loop(0, 24)
    def _zero(k):
        pltpu.sync_copy(hv[0], agg_sh.at[pl.ds(rbase + k * K, K)])

    pltpu.sync_copy(hv[0].at[pl.ds(0, 56)],
                    agg_sh.at[pl.ds(rbase + 24 * K, 56)])

    plsc.subcore_barrier()

    for i in range(nslot):
        t = wid + NW * i
        pltpu.sync_copy(sd_hbm.at[t], idxv.at[i])
        _start_gathers(a_hbm, b_hbm, c_hbm, idxv.at[i], asv[i], bdv[i],
                       cv[i], gsem[i], t)

    @pl.loop(0, niter)
    def _iter(j):
        for i in range(nslot):
            t = wid + NW * (nslot * j + i)
            _wait_gathers(a_hbm, b_hbm, c_hbm, idxv.at[i], asv[i], bdv[i],
                          cv[i], gsem[i], t)

            @pl.when(j > 0)
            def _wait_prev_scatter():
                pltpu.make_async_copy(hv[i], agg_sh.at[dsc[i]], ssem[i]).wait()

            _compute_h(asv[i], bdv[i], cv[i], hv[i], False)
            _snapshot_dst(idxv.at[i], dsc[i])
            pltpu.async_copy(hv[i], agg_sh.at[dsc[i]], ssem[i], add=True)

            @pl.when(j < niter - 1)
            def _prefetch():
                t2 = wid + NW * (nslot * (j + 1) + i)
                pltpu.sync_copy(sd_hbm.at[t2], idxv.at[i])
                _start_gathers(a_hbm, b_hbm, c_hbm, idxv.at[i], asv[i],
                               bdv[i], cv[i], gsem[i], t2)

    for i in range(nslot):
        pltpu.make_async_copy(hv[i], agg_sh.at[dsc[i]], ssem[i]).wait()

    plsc.subcore_barrier()

    @pl.loop(0, 24)
    def _drain(k):
        o = rbase + k * K
        pltpu.sync_copy(agg_sh.at[pl.ds(o, K)], hv[0])
        pltpu.sync_copy(hv[0], out_hbm.at[c, pl.ds(o, K)])

    o = rbase + 24 * K
    pltpu.sync_copy(agg_sh.at[pl.ds(o, 56)], hv[0].at[pl.ds(0, 56)])
    pltpu.sync_copy(hv[0].at[pl.ds(0, 56)], out_hbm.at[c, pl.ds(o, 56)])


_agg_kernel = functools.partial(
    pl.kernel,
    out_type=jax.ShapeDtypeStruct((NC, NPAD_A, H), jnp.float32),
    mesh=_mesh,
    compiler_params=_sc_params,
    scratch_types=_slot_scratch(NSLOT_AGG, False) + [
        pltpu.VMEM_SHARED((NPAD_A, H), jnp.float32),
    ],
)(_agg_body)


def _h1_body(a_hbm, b_hbm, c_hbm, sd_hbm, out_hbm,
             idxv, dsc, asv, bdv, cv, hv, gsem, ssem):
    c = lax.axis_index("c")
    s = lax.axis_index("s")
    wid = s * NC + c
    nslot = NSLOT_H1
    niter = CPW // nslot

    for i in range(nslot):
        t = wid + NW * i
        pltpu.sync_copy(sd_hbm.at[t], idxv.at[i])
        _start_gathers(a_hbm, b_hbm, c_hbm, idxv.at[i], asv[i], bdv[i],
                       cv[i], gsem[i], t)

    @pl.loop(0, niter)
    def _iter(j):
        for i in range(nslot):
            t = wid + NW * (nslot * j + i)
            _wait_gathers(a_hbm, b_hbm, c_hbm, idxv.at[i], asv[i], bdv[i],
                          cv[i], gsem[i], t)

            @pl.when(j > 0)
            def _wait_prev_store():
                pltpu.make_async_copy(hv[i], _c4_slice(out_hbm, t),
                                      ssem[i]).wait()

            _compute_h(asv[i], bdv[i], cv[i], hv[i], True)
            pltpu.async_copy(hv[i], _c4_slice(out_hbm, t), ssem[i])

            @pl.when(j < niter - 1)
            def _prefetch():
                t2 = wid + NW * (nslot * (j + 1) + i)
                pltpu.sync_copy(sd_hbm.at[t2], idxv.at[i])
                _start_gathers(a_hbm, b_hbm, c_hbm, idxv.at[i], asv[i],
                               bdv[i], cv[i], gsem[i], t2)

    for i in range(nslot):
        pltpu.make_async_copy(hv[i], _c4_slice(out_hbm, 0), ssem[i]).wait()


_h1_kernel = functools.partial(
    pl.kernel,
    out_type=jax.ShapeDtypeStruct((M4, 128), jnp.float32),
    mesh=_mesh,
    compiler_params=_sc_params,
    scratch_types=_slot_scratch(NSLOT_H1, True),
)(_h1_body)


BN = 2000


def _pack_bf16_pair(af):
    bits = jax.lax.bitcast_convert_type(af, jnp.int32)
    rtne = bits + 0x7FFF + ((bits >> 16) & 1)
    lo = (rtne[:, :16] >> 16) & 0xFFFF
    hi = rtne[:, 16:] & jnp.int32(-65536)
    return hi | lo
BEC = 7168


def _embed_body(nf_ref, we_ref, be_ref, w1a_ref, w1b_ref, x_ref, a_ref, b_ref):
    x = jnp.maximum(
        jnp.dot(nf_ref[...], we_ref[...], preferred_element_type=jnp.float32)
        + be_ref[...], 0.0)
    x_ref[...] = x
    a_ref[...] = _pack_bf16_pair(
        jnp.dot(x, w1a_ref[...], preferred_element_type=jnp.float32))
    b_ref[...] = _pack_bf16_pair(
        jnp.dot(x, w1b_ref[...], preferred_element_type=jnp.float32))


def _embed_call(nf, we, be2, w1a, w1b):
    n32 = jax.ShapeDtypeStruct((N, D), jnp.float32)
    nbf = jax.ShapeDtypeStruct((N, 16), jnp.int32)
    return pl.pallas_call(
        _embed_body,
        grid=(N // BN,),
        in_specs=[
            pl.BlockSpec((BN, 7), lambda i: (i, 0)),
            pl.BlockSpec((7, D), lambda i: (0, 0)),
            pl.BlockSpec((1, D), lambda i: (0, 0)),
            pl.BlockSpec((D, D), lambda i: (0, 0)),
            pl.BlockSpec((D, D), lambda i: (0, 0)),
        ],
        out_specs=[pl.BlockSpec((BN, D), lambda i: (i, 0)),
                   pl.BlockSpec((BN, 16), lambda i: (i, 0)),
                   pl.BlockSpec((BN, 16), lambda i: (i, 0))],
        out_shape=[n32, nbf, nbf],
    )(nf, we, be2, w1a, w1b)


def _edgec_body(*refs):
    nout = (len(refs) - 6)
    ea_refs, w_ref, b_ref, crefs = refs[:4], refs[4], refs[5], refs[6:]
    eas = [r[...] for r in ea_refs]
    i = pl.program_id(0)
    iota = lax.broadcasted_iota(jnp.int32, (BE4, 1), 0)
    valids = [(4 * i + j) * BE4 + iota < E for j in range(4)]
    for l, cref in enumerate(crefs):
        parts = []
        for j in range(4):
            v = jnp.dot(eas[j], w_ref[l],
                        preferred_element_type=jnp.float32) + b_ref[l]
            parts.append(jnp.where(valids[j], v, -1e30))
        cref[...] = jnp.concatenate(parts, axis=1)


def _edgec_call(ea, wstack, bstack):
    nout = wstack.shape[0]
    e32 = jax.ShapeDtypeStruct((M4, 128), jnp.float32)
    last = (E - 1) // BE4

    def easpec(j):
        return pl.BlockSpec(
            (BE4, 16), lambda i, j=j: (jnp.minimum(4 * i + j, last), 0))

    return pl.pallas_call(
        _edgec_body,
        grid=(GE,),
        in_specs=[
            easpec(0), easpec(1), easpec(2), easpec(3),
            pl.BlockSpec((nout, 16, H), lambda i: (0, 0, 0)),
            pl.BlockSpec((nout, H), lambda i: (0, 0)),
        ],
        out_specs=[pl.BlockSpec((BE4, 128), lambda i: (i, 0))] * nout,
        out_shape=[e32] * nout,
    )(ea, ea, ea, ea, wstack, bstack)


def _update_body(x_ref, agg2_ref, cnt2_ref, wm2_ref, bm2_ref, wua_ref, wub_ref,
                 bu_ref, g_ref, be_ref, wna_ref, wnb_ref,
                 xo_ref, ao_ref, bo_ref):
    x = x_ref[...]
    aggs = agg2_ref[0] + agg2_ref[1]
    cnt = cnt2_ref[0] + cnt2_ref[1]
    msum = jnp.dot(aggs, wm2_ref[...], preferred_element_type=jnp.float32) \
        + cnt * bm2_ref[...]
    agg = msum / (cnt + 1e-6)
    upd = jnp.maximum(
        jnp.dot(x, wua_ref[...], preferred_element_type=jnp.float32)
        + jnp.dot(agg, wub_ref[...], preferred_element_type=jnp.float32)
        + bu_ref[...], 0.0)
    xn = x + upd
    mu = jnp.mean(xn, axis=-1, keepdims=True)
    var = jnp.mean((xn - mu) ** 2, axis=-1, keepdims=True)
    xn = (xn - mu) / jnp.sqrt(var + 1e-5) * g_ref[...] + be_ref[...]
    xo_ref[...] = xn
    ao_ref[...] = _pack_bf16_pair(
        jnp.dot(xn, wna_ref[...], preferred_element_type=jnp.float32))
    bo_ref[...] = _pack_bf16_pair(
        jnp.dot(xn, wnb_ref[...], preferred_element_type=jnp.float32))


def _update_call(x, agg2, cnt2, wm2, bm2_2, wua, wub, bu2, g2, be2, wna, wnb):
    n32 = jax.ShapeDtypeStruct((N, D), jnp.float32)
    nbf = jax.ShapeDtypeStruct((N, 16), jnp.int32)
    wspec = pl.BlockSpec((D, D), lambda i: (0, 0))
    vspec = pl.BlockSpec((1, D), lambda i: (0, 0))
    return pl.pallas_call(
        _update_body,
        grid=(N // BN,),
        in_specs=[
            pl.BlockSpec((BN, D), lambda i: (i, 0)),
            pl.BlockSpec((NC, BN, H), lambda i: (0, i, 0)),
            pl.BlockSpec((NC, BN, 1), lambda i: (0, i, 0)),
            wspec, vspec, wspec, wspec, vspec, vspec, vspec, wspec, wspec,
        ],
        out_specs=[pl.BlockSpec((BN, D), lambda i: (i, 0)),
                   pl.BlockSpec((BN, 16), lambda i: (i, 0)),
                   pl.BlockSpec((BN, 16), lambda i: (i, 0))],
        out_shape=[n32, nbf, nbf],
    )(x, agg2, cnt2, wm2, bm2_2, wua, wub, bu2, g2, be2, wna, wnb)


def _pred_body(h4_ref, wp2_ref, bp2_ref, wp3_ref, bp3_ref, out_ref):
    h2 = jnp.maximum(
        jnp.dot(h4_ref[...], wp2_ref[...], preferred_element_type=jnp.float32)
        + bp2_ref[...], 0.0)
    z = jnp.dot(h2, wp3_ref[...], preferred_element_type=jnp.float32) \
        + bp3_ref[...]
    out_ref[...] = jnp.tanh(z)


def _pred_call(h4, wp2_4, bp2_4, wp3_4, bp3_4):
    return pl.pallas_call(
        _pred_body,
        grid=(GE,),
        in_specs=[
            pl.BlockSpec((BE4, 128), lambda i: (i, 0)),
            pl.BlockSpec((128, 64), lambda i: (0, 0)),
            pl.BlockSpec((1, 64), lambda i: (0, 0)),
            pl.BlockSpec((64, 4), lambda i: (0, 0)),
            pl.BlockSpec((1, 4), lambda i: (0, 0)),
        ],
        out_specs=pl.BlockSpec((BE4, 4), lambda i: (i, 0)),
        out_shape=jax.ShapeDtypeStruct((M4, 4), jnp.float32),
    )(h4, wp2_4, bp2_4, wp3_4, bp3_4)


def kernel(node_features, edge_index, edge_attr, W_embed, b_embed,
           Wm1_0, bm1_0, Wm2_0, bm2_0, Wu_0, bu_0, g_0, be_0,
           Wm1_1, bm1_1, Wm2_1, bm2_1, Wu_1, bu_1, g_1, be_1,
           Wm1_2, bm1_2, Wm2_2, bm2_2, Wu_2, bu_2, g_2, be_2,
           Wp1, bp1, Wp2, bp2, Wp3, bp3):
    src = edge_index[0]
    dst = edge_index[1]

    pad = E_PAD - E
    src_p = jnp.concatenate([src, jnp.zeros((pad,), jnp.int32)])
    dst_p = jnp.concatenate([dst, jnp.zeros((pad,), jnp.int32)])
    def _chunk_perm(v):
        return v.reshape(GE, 4, 56, 32).transpose(0, 2, 3, 1).reshape(
            NCHUNK_PAD, K)

    sd = jnp.stack([_chunk_perm(src_p), _chunk_perm(dst_p)], axis=1)

    wm1 = [Wm1_0, Wm1_1, Wm1_2]
    bm1 = [bm1_0, bm1_1, bm1_2]
    wm2 = [Wm2_0, Wm2_1, Wm2_2]
    bm2 = [bm2_0, bm2_1, bm2_2]
    wu = [Wu_0, Wu_1, Wu_2]
    bu = [bu_0, bu_1, bu_2]
    g = [g_0, g_1, g_2]
    be = [be_0, be_1, be_2]

    cnt2 = _count_kernel(dst).reshape(NC, NPAD, 1)

    (c0,) = _edgec_call(edge_attr, wm1[0][2 * D:][None], bm1[0][None])
    wstack = jnp.stack([wm1[1][2 * D:], wm1[2][2 * D:], Wp1[2 * D:]])
    bstack = jnp.stack([bm1[1], bm1[2], bp1])
    c123 = _edgec_call(edge_attr, wstack, bstack)
    c_all = [c0] + list(c123)

    x, a, b = _embed_call(node_features, W_embed, b_embed[None, :],
                          wm1[0][:D], wm1[0][D:2 * D])

    for l in range(3):
        agg2 = _agg_kernel(a, b, c_all[l], sd)
        if l < 2:
            wna, wnb = wm1[l + 1][:D], wm1[l + 1][D:2 * D]
        else:
            wna, wnb = Wp1[:D], Wp1[D:2 * D]
        x, a, b = _update_call(
            x, agg2, cnt2, wm2[l], bm2[l][None, :], wu[l][:D], wu[l][D:],
            bu[l][None, :], g[l][None, :], be[l][None, :], wna, wnb)

    zb2 = jnp.zeros((D, H // 2), jnp.float32)
    wp2_4 = jnp.concatenate([
        jnp.concatenate([Wp2 if r == c else zb2 for c in range(4)], axis=1)
        for r in range(4)], axis=0)
    zb3 = jnp.zeros((H // 2, 1), jnp.float32)
    wp3_4 = jnp.concatenate([
        jnp.concatenate([Wp3 if r == c else zb3 for c in range(4)], axis=1)
        for r in range(4)], axis=0)
    bp2_4 = jnp.tile(bp2, 4)[None, :]
    bp3_4 = jnp.tile(bp3, 4)[None, :]

    h4 = _h1_kernel(a, b, c_all[3], sd)
    p4 = _pred_call(h4, wp2_4, bp2_4, wp3_4, bp3_4)
    p = p4.reshape(GE, BE4, 4).transpose(0, 2, 1).reshape(E_PAD)
    return p[:E]


# speedup vs baseline: 6.2002x; 1.0056x over previous
import functools

import jax
import jax.numpy as jnp
from jax import lax
from jax.experimental import pallas as pl
from jax.experimental.pallas import tpu as pltpu
from jax.experimental.pallas import tpu_sc as plsc

N = 50000
E = 800000
D = 32
H = 32

NC = 2
NS = 16

KC = 128
NCHUNK = E // KC
NW = NC * NS
K = 128
CPW = 196
NCHUNK_PAD = NW * CPW
E_PAD = NCHUNK_PAD * K
NPAD = 51200
RPS = NPAD // NS
NPAD_A = 50048
RPA = NPAD_A // NS
M4 = E_PAD // 4
BE4 = 1792
GE = E_PAD // (4 * BE4)

_mesh = plsc.VectorSubcoreMesh(core_axis_name="c", subcore_axis_name="s")
_sc_params = pltpu.CompilerParams(use_tc_tiling_on_sc=False)


def _cnt_body(dst_hbm, out_hbm, dstv, onesv, zb, cnt_sh, sem):
    c = lax.axis_index("c")
    s = lax.axis_index("s")
    wid = s * NC + c

    @pl.loop(0, 8)
    def _fill_ones(i):
        onesv[pl.ds(i * 16, 16)] = jnp.full((16,), 1.0, jnp.float32)

    @pl.loop(0, 50)
    def _fill_zeros(i):
        zb[pl.ds(i * 16, 16)] = jnp.zeros((16,), jnp.float32)

    base = s * RPS

    @pl.loop(0, RPS // 800)
    def _zero(k):
        pltpu.sync_copy(zb, cnt_sh.at[pl.ds(base + k * 800, 800)])

    plsc.subcore_barrier()

    @pl.loop(wid, NCHUNK, step=NW)
    def _scatter(t):
        eb = t * KC
        pltpu.sync_copy(dst_hbm.at[pl.ds(eb, KC)], dstv)
        pltpu.sync_copy(onesv, cnt_sh.at[dstv], add=True)

    plsc.subcore_barrier()

    @pl.loop(0, RPS // 800)
    def _drain(k):
        o = base + k * 800
        pltpu.sync_copy(cnt_sh.at[pl.ds(o, 800)], zb)
        pltpu.sync_copy(zb, out_hbm.at[pl.ds(c * NPAD + o, 800)])


_count_kernel = functools.partial(
    pl.kernel,
    out_type=jax.ShapeDtypeStruct((NC * NPAD,), jnp.float32),
    mesh=_mesh,
    compiler_params=_sc_params,
    scratch_types=[
        pltpu.VMEM((KC,), jnp.int32),
        pltpu.VMEM((KC,), jnp.float32),
        pltpu.VMEM((800,), jnp.float32),
        pltpu.VMEM_SHARED((NPAD,), jnp.float32),
        pltpu.SemaphoreType.DMA,
    ],
)(_cnt_body)


NSLOT_AGG = 2
NSLOT_H1 = 4


def _unpack_cols(w):
    lo = jax.lax.bitcast_convert_type(w << 16, jnp.float32)
    hi = jax.lax.bitcast_convert_type(w & jnp.int32(-65536), jnp.float32)
    return lo, hi


def _compute_h(asv, bdv, cv, hv, packed_out):
    @pl.loop(0, K, unroll=8)
    def _rows(i):
        r = i // 4
        cb = (i % 4) * 32
        a01 = _unpack_cols(asv[i, pl.ds(0, 16)])
        b01 = _unpack_cols(bdv[i, pl.ds(0, 16)])
        for half, j in ((0, 0), (1, 16)):
            v = jnp.maximum(
                a01[half] + b01[half] + cv[r, pl.ds(cb + j, 16)], 0.0)
            if packed_out:
                hv[r, pl.ds(cb + j, 16)] = v
            else:
                hv[i, pl.ds(j, 16)] = v


def _snapshot_dst(iv, dsc):
    @pl.loop(0, K // 16)
    def _cp(q):
        dsc[pl.ds(q * 16, 16)] = iv[1, pl.ds(q * 16, 16)]


def _c4_slice(c_hbm, t):
    return c_hbm.at[pl.ds((t // 56) * BE4 + (t % 56) * 32, 32)]


def _start_gathers(a_hbm, b_hbm, c_hbm, iv, asv, bdv, cv, gsem, t):
    pltpu.async_copy(a_hbm.at[iv.at[0]], asv, gsem)
    pltpu.async_copy(b_hbm.at[iv.at[1]], bdv, gsem)
    pltpu.async_copy(_c4_slice(c_hbm, t), cv, gsem)


def _wait_gathers(a_hbm, b_hbm, c_hbm, iv, asv, bdv, cv, gsem, t):
    pltpu.make_async_copy(a_hbm.at[iv.at[0]], asv, gsem).wait()
    pltpu.make_async_copy(b_hbm.at[iv.at[1]], bdv, gsem).wait()
    pltpu.make_async_copy(_c4_slice(c_hbm, t), cv, gsem).wait()


def _slot_scratch(nslot, packed_hv):
    hv_shape = (32, 128) if packed_hv else (K, H)
    return [
        pltpu.VMEM((nslot, 2, K), jnp.int32),
        [pltpu.VMEM((K,), jnp.int32) for _ in range(nslot)],
        [pltpu.VMEM((K, 16), jnp.int32) for _ in range(nslot)],
        [pltpu.VMEM((K, 16), jnp.int32) for _ in range(nslot)],
        [pltpu.VMEM((32, 128), jnp.float32) for _ in range(nslot)],
        [pltpu.VMEM(hv_shape, jnp.float32) for _ in range(nslot)],
        [pltpu.SemaphoreType.DMA for _ in range(nslot)],
        [pltpu.SemaphoreType.DMA for _ in range(nslot)],
    ]


def _agg_body(a_hbm, b_hbm, c_hbm, sd_hbm, out_hbm,
              idxv, dsc, asv, bdv, cv, hv, gsem, ssem, agg_sh):
    c = lax.axis_index("c")
    s = lax.axis_index("s")
    wid = s * NC + c
    nslot = NSLOT_AGG
    niter = CPW // nslot

    @pl.loop(0, K * 2)
    def _fill_zeros(q):
        hv[0][q // 2, pl.ds((q % 2) * 16, 16)] = jnp.zeros((16,), jnp.float32)

    rbase = s * RPA

    for k in range(24):
        pltpu.async_copy(hv[0], agg_sh.at[pl.ds(rbase + k * K, K)], gsem[0])
    pltpu.async_copy(hv[0].at[pl.ds(0, 56)],
                     agg_sh.at[pl.ds(rbase + 24 * K, 56)], gsem[0])
    for k in range(24):
        pltpu.make_async_copy(hv[0], agg_sh.at[pl.ds(rbase + k * K, K)],
                              gsem[0]).wait()
    pltpu.make_async_copy(hv[0].at[pl.ds(0, 56)],
                          agg_sh.at[pl.ds(rbase + 24 * K, 56)],
                          gsem[0]).wait()

    plsc.subcore_barrier()

    for i in range(nslot):
        t = wid + NW * i
        pltpu.sync_copy(sd_hbm.at[t], idxv.at[i])
        _start_gathers(a_hbm, b_hbm, c_hbm, idxv.at[i], asv[i], bdv[i],
                       cv[i], gsem[i], t)

    @pl.loop(0, niter)
    def _iter(j):
        for i in range(nslot):
            t = wid + NW * (nslot * j + i)
            _wait_gathers(a_hbm, b_hbm, c_hbm, idxv.at[i], asv[i], bdv[i],
                          cv[i], gsem[i], t)

            @pl.when(j > 0)
            def _wait_prev_scatter():
                pltpu.make_async_copy(hv[i], agg_sh.at[dsc[i]], ssem[i]).wait()

            _compute_h(asv[i], bdv[i], cv[i], hv[i], False)
            _snapshot_dst(idxv.at[i], dsc[i])
            pltpu.async_copy(hv[i], agg_sh.at[dsc[i]], ssem[i], add=True)

            @pl.when(j < niter - 1)
            def _prefetch():
                t2 = wid + NW * (nslot * (j + 1) + i)
                pltpu.sync_copy(sd_hbm.at[t2], idxv.at[i])
                _start_gathers(a_hbm, b_hbm, c_hbm, idxv.at[i], asv[i],
                               bdv[i], cv[i], gsem[i], t2)

    for i in range(nslot):
        pltpu.make_async_copy(hv[i], agg_sh.at[dsc[i]], ssem[i]).wait()

    plsc.subcore_barrier()

    for k in range(24):
        b = hv[k % 2]
        o = rbase + k * K
        if k >= 2:
            po = rbase + (k - 2) * K
            pltpu.make_async_copy(b, out_hbm.at[c, pl.ds(po, K)],
                                  ssem[k % 2]).wait()
        pltpu.sync_copy(agg_sh.at[pl.ds(o, K)], b)
        pltpu.async_copy(b, out_hbm.at[c, pl.ds(o, K)], ssem[k % 2])

    pltpu.make_async_copy(hv[0], out_hbm.at[c, pl.ds(rbase + 22 * K, K)],
                          ssem[0]).wait()
    o = rbase + 24 * K
    pltpu.sync_copy(agg_sh.at[pl.ds(o, 56)], hv[0].at[pl.ds(0, 56)])
    pltpu.async_copy(hv[0].at[pl.ds(0, 56)], out_hbm.at[c, pl.ds(o, 56)],
                     ssem[0])
    pltpu.make_async_copy(hv[0].at[pl.ds(0, 56)],
                          out_hbm.at[c, pl.ds(o, 56)], ssem[0]).wait()
    pltpu.make_async_copy(hv[1], out_hbm.at[c, pl.ds(rbase + 23 * K, K)],
                          ssem[1]).wait()


_agg_kernel = functools.partial(
    pl.kernel,
    out_type=jax.ShapeDtypeStruct((NC, NPAD_A, H), jnp.float32),
    mesh=_mesh,
    compiler_params=_sc_params,
    scratch_types=_slot_scratch(NSLOT_AGG, False) + [
        pltpu.VMEM_SHARED((NPAD_A, H), jnp.float32),
    ],
)(_agg_body)


def _h1_body(a_hbm, b_hbm, c_hbm, sd_hbm, out_hbm,
             idxv, dsc, asv, bdv, cv, hv, gsem, ssem):
    c = lax.axis_index("c")
    s = lax.axis_index("s")
    wid = s * NC + c
    nslot = NSLOT_H1
    niter = CPW // nslot

    for i in range(nslot):
        t = wid + NW * i
        pltpu.sync_copy(sd_hbm.at[t], idxv.at[i])
        _start_gathers(a_hbm, b_hbm, c_hbm, idxv.at[i], asv[i], bdv[i],
                       cv[i], gsem[i], t)

    @pl.loop(0, niter)
    def _iter(j):
        for i in range(nslot):
            t = wid + NW * (nslot * j + i)
            _wait_gathers(a_hbm, b_hbm, c_hbm, idxv.at[i], asv[i], bdv[i],
                          cv[i], gsem[i], t)

            @pl.when(j > 0)
            def _wait_prev_store():
                pltpu.make_async_copy(hv[i], _c4_slice(out_hbm, t),
                                      ssem[i]).wait()

            _compute_h(asv[i], bdv[i], cv[i], hv[i], True)
            pltpu.async_copy(hv[i], _c4_slice(out_hbm, t), ssem[i])

            @pl.when(j < niter - 1)
            def _prefetch():
                t2 = wid + NW * (nslot * (j + 1) + i)
                pltpu.sync_copy(sd_hbm.at[t2], idxv.at[i])
                _start_gathers(a_hbm, b_hbm, c_hbm, idxv.at[i], asv[i],
                               bdv[i], cv[i], gsem[i], t2)

    for i in range(nslot):
        pltpu.make_async_copy(hv[i], _c4_slice(out_hbm, 0), ssem[i]).wait()


_h1_kernel = functools.partial(
    pl.kernel,
    out_type=jax.ShapeDtypeStruct((M4, 128), jnp.float32),
    mesh=_mesh,
    compiler_params=_sc_params,
    scratch_types=_slot_scratch(NSLOT_H1, True),
)(_h1_body)


BN = 2000


def _pack_bf16_pair(af):
    bits = jax.lax.bitcast_convert_type(af, jnp.int32)
    rtne = bits + 0x7FFF + ((bits >> 16) & 1)
    lo = (rtne[:, :16] >> 16) & 0xFFFF
    hi = rtne[:, 16:] & jnp.int32(-65536)
    return hi | lo
BEC = 7168


def _embed_body(nf_ref, we_ref, be_ref, w1a_ref, w1b_ref, x_ref, a_ref, b_ref):
    x = jnp.maximum(
        jnp.dot(nf_ref[...], we_ref[...], preferred_element_type=jnp.float32)
        + be_ref[...], 0.0)
    x_ref[...] = x
    a_ref[...] = _pack_bf16_pair(
        jnp.dot(x, w1a_ref[...], preferred_element_type=jnp.float32))
    b_ref[...] = _pack_bf16_pair(
        jnp.dot(x, w1b_ref[...], preferred_element_type=jnp.float32))


def _embed_call(nf, we, be2, w1a, w1b):
    n32 = jax.ShapeDtypeStruct((N, D), jnp.float32)
    nbf = jax.ShapeDtypeStruct((N, 16), jnp.int32)
    return pl.pallas_call(
        _embed_body,
        grid=(N // BN,),
        in_specs=[
            pl.BlockSpec((BN, 7), lambda i: (i, 0)),
            pl.BlockSpec((7, D), lambda i: (0, 0)),
            pl.BlockSpec((1, D), lambda i: (0, 0)),
            pl.BlockSpec((D, D), lambda i: (0, 0)),
            pl.BlockSpec((D, D), lambda i: (0, 0)),
        ],
        out_specs=[pl.BlockSpec((BN, D), lambda i: (i, 0)),
                   pl.BlockSpec((BN, 16), lambda i: (i, 0)),
                   pl.BlockSpec((BN, 16), lambda i: (i, 0))],
        out_shape=[n32, nbf, nbf],
    )(nf, we, be2, w1a, w1b)


def _edgec_body(*refs):
    nout = (len(refs) - 6)
    ea_refs, w_ref, b_ref, crefs = refs[:4], refs[4], refs[5], refs[6:]
    eas = [r[...] for r in ea_refs]
    i = pl.program_id(0)
    iota = lax.broadcasted_iota(jnp.int32, (BE4, 1), 0)
    valids = [(4 * i + j) * BE4 + iota < E for j in range(4)]
    for l, cref in enumerate(crefs):
        parts = []
        for j in range(4):
            v = jnp.dot(eas[j], w_ref[l],
                        preferred_element_type=jnp.float32) + b_ref[l]
            parts.append(jnp.where(valids[j], v, -1e30))
        cref[...] = jnp.concatenate(parts, axis=1)


def _edgec_call(ea, wstack, bstack):
    nout = wstack.shape[0]
    e32 = jax.ShapeDtypeStruct((M4, 128), jnp.float32)
    last = (E - 1) // BE4

    def easpec(j):
        return pl.BlockSpec(
            (BE4, 16), lambda i, j=j: (jnp.minimum(4 * i + j, last), 0))

    return pl.pallas_call(
        _edgec_body,
        grid=(GE,),
        in_specs=[
            easpec(0), easpec(1), easpec(2), easpec(3),
            pl.BlockSpec((nout, 16, H), lambda i: (0, 0, 0)),
            pl.BlockSpec((nout, H), lambda i: (0, 0)),
        ],
        out_specs=[pl.BlockSpec((BE4, 128), lambda i: (i, 0))] * nout,
        out_shape=[e32] * nout,
    )(ea, ea, ea, ea, wstack, bstack)


def _update_body(x_ref, agg2_ref, cnt2_ref, wm2_ref, bm2_ref, wua_ref, wub_ref,
                 bu_ref, g_ref, be_ref, wna_ref, wnb_ref,
                 xo_ref, ao_ref, bo_ref):
    x = x_ref[...]
    aggs = agg2_ref[0] + agg2_ref[1]
    cnt = cnt2_ref[0] + cnt2_ref[1]
    msum = jnp.dot(aggs, wm2_ref[...], preferred_element_type=jnp.float32) \
        + cnt * bm2_ref[...]
    agg = msum / (cnt + 1e-6)
    upd = jnp.maximum(
        jnp.dot(x, wua_ref[...], preferred_element_type=jnp.float32)
        + jnp.dot(agg, wub_ref[...], preferred_element_type=jnp.float32)
        + bu_ref[...], 0.0)
    xn = x + upd
    mu = jnp.mean(xn, axis=-1, keepdims=True)
    var = jnp.mean((xn - mu) ** 2, axis=-1, keepdims=True)
    xn = (xn - mu) / jnp.sqrt(var + 1e-5) * g_ref[...] + be_ref[...]
    xo_ref[...] = xn
    ao_ref[...] = _pack_bf16_pair(
        jnp.dot(xn, wna_ref[...], preferred_element_type=jnp.float32))
    bo_ref[...] = _pack_bf16_pair(
        jnp.dot(xn, wnb_ref[...], preferred_element_type=jnp.float32))


def _update_call(x, agg2, cnt2, wm2, bm2_2, wua, wub, bu2, g2, be2, wna, wnb):
    n32 = jax.ShapeDtypeStruct((N, D), jnp.float32)
    nbf = jax.ShapeDtypeStruct((N, 16), jnp.int32)
    wspec = pl.BlockSpec((D, D), lambda i: (0, 0))
    vspec = pl.BlockSpec((1, D), lambda i: (0, 0))
    return pl.pallas_call(
        _update_body,
        grid=(N // BN,),
        in_specs=[
            pl.BlockSpec((BN, D), lambda i: (i, 0)),
            pl.BlockSpec((NC, BN, H), lambda i: (0, i, 0)),
            pl.BlockSpec((NC, BN, 1), lambda i: (0, i, 0)),
            wspec, vspec, wspec, wspec, vspec, vspec, vspec, wspec, wspec,
        ],
        out_specs=[pl.BlockSpec((BN, D), lambda i: (i, 0)),
                   pl.BlockSpec((BN, 16), lambda i: (i, 0)),
                   pl.BlockSpec((BN, 16), lambda i: (i, 0))],
        out_shape=[n32, nbf, nbf],
    )(x, agg2, cnt2, wm2, bm2_2, wua, wub, bu2, g2, be2, wna, wnb)


def _pred_body(h4_ref, wp2_ref, bp2_ref, wp3_ref, bp3_ref, out_ref):
    h2 = jnp.maximum(
        jnp.dot(h4_ref[...], wp2_ref[...], preferred_element_type=jnp.float32)
        + bp2_ref[...], 0.0)
    z = jnp.dot(h2, wp3_ref[...], preferred_element_type=jnp.float32) \
        + bp3_ref[...]
    out_ref[...] = jnp.tanh(z)


def _pred_call(h4, wp2_4, bp2_4, wp3_4, bp3_4):
    return pl.pallas_call(
        _pred_body,
        grid=(GE,),
        in_specs=[
            pl.BlockSpec((BE4, 128), lambda i: (i, 0)),
            pl.BlockSpec((128, 64), lambda i: (0, 0)),
            pl.BlockSpec((1, 64), lambda i: (0, 0)),
            pl.BlockSpec((64, 4), lambda i: (0, 0)),
            pl.BlockSpec((1, 4), lambda i: (0, 0)),
        ],
        out_specs=pl.BlockSpec((BE4, 4), lambda i: (i, 0)),
        out_shape=jax.ShapeDtypeStruct((M4, 4), jnp.float32),
    )(h4, wp2_4, bp2_4, wp3_4, bp3_4)


def kernel(node_features, edge_index, edge_attr, W_embed, b_embed,
           Wm1_0, bm1_0, Wm2_0, bm2_0, Wu_0, bu_0, g_0, be_0,
           Wm1_1, bm1_1, Wm2_1, bm2_1, Wu_1, bu_1, g_1, be_1,
           Wm1_2, bm1_2, Wm2_2, bm2_2, Wu_2, bu_2, g_2, be_2,
           Wp1, bp1, Wp2, bp2, Wp3, bp3):
    src = edge_index[0]
    dst = edge_index[1]

    pad = E_PAD - E
    src_p = jnp.concatenate([src, jnp.zeros((pad,), jnp.int32)])
    dst_p = jnp.concatenate([dst, jnp.zeros((pad,), jnp.int32)])
    def _chunk_perm(v):
        return v.reshape(GE, 4, 56, 32).transpose(0, 2, 3, 1).reshape(
            NCHUNK_PAD, K)

    sd = jnp.stack([_chunk_perm(src_p), _chunk_perm(dst_p)], axis=1)

    wm1 = [Wm1_0, Wm1_1, Wm1_2]
    bm1 = [bm1_0, bm1_1, bm1_2]
    wm2 = [Wm2_0, Wm2_1, Wm2_2]
    bm2 = [bm2_0, bm2_1, bm2_2]
    wu = [Wu_0, Wu_1, Wu_2]
    bu = [bu_0, bu_1, bu_2]
    g = [g_0, g_1, g_2]
    be = [be_0, be_1, be_2]

    cnt2 = _count_kernel(dst).reshape(NC, NPAD, 1)

    (c0,) = _edgec_call(edge_attr, wm1[0][2 * D:][None], bm1[0][None])
    wstack = jnp.stack([wm1[1][2 * D:], wm1[2][2 * D:], Wp1[2 * D:]])
    bstack = jnp.stack([bm1[1], bm1[2], bp1])
    c123 = _edgec_call(edge_attr, wstack, bstack)
    c_all = [c0] + list(c123)

    x, a, b = _embed_call(node_features, W_embed, b_embed[None, :],
                          wm1[0][:D], wm1[0][D:2 * D])

    for l in range(3):
        agg2 = _agg_kernel(a, b, c_all[l], sd)
        if l < 2:
            wna, wnb = wm1[l + 1][:D], wm1[l + 1][D:2 * D]
        else:
            wna, wnb = Wp1[:D], Wp1[D:2 * D]
        x, a, b = _update_call(
            x, agg2, cnt2, wm2[l], bm2[l][None, :], wu[l][:D], wu[l][D:],
            bu[l][None, :], g[l][None, :], be[l][None, :], wna, wnb)

    zb2 = jnp.zeros((D, H // 2), jnp.float32)
    wp2_4 = jnp.concatenate([
        jnp.concatenate([Wp2 if r == c else zb2 for c in range(4)], axis=1)
        for r in range(4)], axis=0)
    zb3 = jnp.zeros((H // 2, 1), jnp.float32)
    wp3_4 = jnp.concatenate([
        jnp.concatenate([Wp3 if r == c else zb3 for c in range(4)], axis=1)
        for r in range(4)], axis=0)
    bp2_4 = jnp.tile(bp2, 4)[None, :]
    bp3_4 = jnp.tile(bp3, 4)[None, :]

    h4 = _h1_kernel(a, b, c_all[3], sd)
    p4 = _pred_call(h4, wp2_4, bp2_4, wp3_4, bp3_4)
    p = p4.reshape(GE, BE4, 4).transpose(0, 2, 1).reshape(E_PAD)
    return p[:E]
